# baseline (speedup 1.0000x reference)
import functools

import jax
import jax.numpy as jnp
from jax import lax
from jax.experimental import pallas as pl
from jax.experimental.pallas import tpu as pltpu
from jax.experimental.pallas import tpu_sc as plsc

N = 10000
E = 320000
D = 128
NP = 10240
NW = 32
EPW = E // NW
EC = 80
NCHUNK = EPW // EC
ROWS_PER_TILE = NP // 16
ZB = 128


def _edge_agg(x, src, dst):
    mesh = plsc.VectorSubcoreMesh(core_axis_name="c", subcore_axis_name="s")

    @functools.partial(
        pl.kernel,
        out_type=[
            jax.ShapeDtypeStruct((2, NP, D), jnp.float32),
            jax.ShapeDtypeStruct((2, 2, N), jnp.float32),
        ],
        mesh=mesh,
        scratch_types=[
            pltpu.VMEM((EC,), jnp.int32),
            pltpu.VMEM((EC,), jnp.int32),
            pltpu.VMEM((EC,), jnp.int32),
            pltpu.VMEM((EC,), jnp.int32),
            pltpu.VMEM((EC,), jnp.int32),
            pltpu.VMEM((EC,), jnp.int32),
            pltpu.VMEM((EC, D), jnp.float32),
            pltpu.VMEM((EC, D), jnp.float32),
            pltpu.VMEM((N,), jnp.float32),
            pltpu.VMEM((N,), jnp.float32),
            pltpu.VMEM_SHARED((NP, D), jnp.float32),
            pltpu.VMEM_SHARED((N,), jnp.float32),
            pltpu.VMEM_SHARED((N,), jnp.float32),
            pltpu.SemaphoreType.DMA,
            pltpu.SemaphoreType.DMA,
            pltpu.SemaphoreType.DMA,
            pltpu.SemaphoreType.DMA,
            pltpu.SemaphoreType.DMA,
            pltpu.SemaphoreType.DMA,
        ],
        compiler_params=pltpu.CompilerParams(needs_layout_passes=False),
    )
    def k(x_hbm, src_hbm, dst_hbm, agg_out, hist_out,
          srcA, dstA, srcB, dstB, dstSA, dstSB, rows_a, rows_b, hd_loc, hs_loc,
          agg_sh, hd_sh, hs_sh, sia, sib, sra, srb, ssa, ssb):
        c = lax.axis_index("c")
        sid = lax.axis_index("s")
        z16 = jnp.zeros((16,), jnp.float32)
        ones16 = jnp.ones((16,), jnp.float32)
        wid = sid * 2 + c

        def start_idx(kk, sv, dv, sem):
            off = pl.multiple_of(wid * EPW + kk * EC, 8)
            pltpu.async_copy(src_hbm.at[pl.ds(off, EC)], sv, sem)
            pltpu.async_copy(dst_hbm.at[pl.ds(off, EC)], dv, sem)

        def wait_idx(sv, dv, sem):
            pltpu.make_async_copy(src_hbm.at[pl.ds(0, EC)], sv, sem).wait()
            pltpu.make_async_copy(dst_hbm.at[pl.ds(0, EC)], dv, sem).wait()

        def start_gather(sv, buf, sem):
            pltpu.async_copy(x_hbm.at[sv], buf, sem)

        def wait_gather(buf, sem):
            pltpu.make_async_copy(x_hbm.at[srcA], buf, sem).wait()

        def hists_and_stage(sv, dv, ds_buf):
            for j in range(EC // 16):
                di = dv[pl.ds(j * 16, 16)]
                si = sv[pl.ds(j * 16, 16)]
                ds_buf[pl.ds(j * 16, 16)] = di
                plsc.addupdate_scatter(hd_loc, [di], ones16)
                plsc.addupdate_scatter(hs_loc, [si], ones16)

        def wait_scatter(rows, ds_buf, sem):
            pltpu.make_async_copy(rows, agg_sh.at[ds_buf], sem).wait()

        start_idx(0, srcA, dstA, sia)
        start_idx(1, srcB, dstB, sib)

        def zloop(i, _):
            hd_loc[pl.ds(i * 16, 16)] = z16
            hs_loc[pl.ds(i * 16, 16)] = z16
            return 0
        lax.fori_loop(0, N // 16, zloop, 0)

        def zloop2(i, _):
            for j in range(D // 16):
                rows_a[i, pl.ds(j * 16, 16)] = z16
            return 0
        lax.fori_loop(0, EC, zloop2, 0)

        r0 = sid * ROWS_PER_TILE
        for t in range(ROWS_PER_TILE // EC):
            pltpu.sync_copy(rows_a, agg_sh.at[pl.ds(r0 + t * EC, EC), :])
        @pl.when(sid == 0)
        def _():
            pltpu.sync_copy(hd_loc, hd_sh)
            pltpu.sync_copy(hs_loc, hs_sh)

        plsc.subcore_barrier()

        wait_idx(srcA, dstA, sia)
        start_gather(srcA, rows_a, sra)

        def body(i, _):
            kb = 2 * i + 1
            wait_gather(rows_a, sra)
            wait_idx(srcB, dstB, sib)
            start_gather(srcB, rows_b, srb)
            hists_and_stage(srcA, dstA, dstSA)
            pltpu.async_copy(rows_a, agg_sh.at[dstSA], ssa, add=True)
            start_idx(kb + 1, srcA, dstA, sia)
            wait_gather(rows_b, srb)
            hists_and_stage(srcB, dstB, dstSB)
            pltpu.async_copy(rows_b, agg_sh.at[dstSB], ssb, add=True)
            @pl.when(kb + 2 < NCHUNK)
            def _():
                start_idx(kb + 2, srcB, dstB, sib)
            wait_idx(srcA, dstA, sia)
            wait_scatter(rows_a, dstSA, ssa)
            start_gather(srcA, rows_a, sra)
            wait_scatter(rows_b, dstSB, ssb)
            return 0

        lax.fori_loop(0, (NCHUNK - 1) // 2, body, 0)
        wait_gather(rows_a, sra)
        hists_and_stage(srcA, dstA, dstSA)
        pltpu.async_copy(rows_a, agg_sh.at[dstSA], ssa, add=True)
        wait_scatter(rows_a, dstSA, ssa)
        plsc.subcore_barrier()

        def merge(e, _):
            off = e * EC
            for j in range(EC // 16):
                srcA[pl.ds(j * 16, 16)] = off + j * 16 + lax.iota(jnp.int32, 16)
            pltpu.sync_copy(hd_loc.at[pl.ds(off, EC)], hd_sh.at[srcA], add=True)
            pltpu.sync_copy(hs_loc.at[pl.ds(off, EC)], hs_sh.at[srcA], add=True)
            return 0

        lax.fori_loop(0, N // EC, merge, 0)
        plsc.subcore_barrier()

        pltpu.sync_copy(agg_sh.at[pl.ds(r0, ROWS_PER_TILE), :],
                        agg_out.at[c, pl.ds(r0, ROWS_PER_TILE), :])
        @pl.when(sid == 0)
        def _():
            pltpu.sync_copy(hd_sh, hist_out.at[c, 0, :])
            pltpu.sync_copy(hs_sh, hist_out.at[c, 1, :])

    return k(x, src, dst)


BN = 1000


def _dense_body(x_ref, a0_ref, a1_ref, cd0_ref, cd1_ref, cs0_ref, cs1_ref,
                wl_ref, bl_ref, wr_ref, wr1_ref, br1_ref, wr2_ref, br2_ref,
                wdel_ref, wc_ref, wres_ref, wd_ref, bd_ref,
                scores_ref, delta_ref, res_ref, ucs_ref):
    i = pl.program_id(0)
    x = x_ref[...]
    cnt = cd0_ref[...] + cd1_ref[...]
    deg = cnt + cs0_ref[...] + cs1_ref[...]
    mean = (a0_ref[...] + a1_ref[...]) / jnp.maximum(cnt, 1.0)

    def mm(a, w_ref):
        return lax.dot_general(a, w_ref[...], (((1,), (1,)), ((), ())),
                               preferred_element_type=jnp.float32)

    pre = mm(mean, wl_ref) + bl_ref[...] + mm(x, wr_ref) + x
    x_gnn = 0.5 * pre * (1.0 + lax.erf(pre * 0.7071067811865476))
    h1 = jnp.maximum(mm(x_gnn, wr1_ref) + br1_ref[...], 0.0)
    sc = jnp.sum(h1 * wr2_ref[...], axis=1, keepdims=True) + br2_ref[0, 0]
    scores_ref[...] = sc + deg

    dpre = mm(x_gnn, wdel_ref)
    cc = mm(x_gnn, wc_ref)
    res_ref[...] = mm(x_gnn, wres_ref)
    z = mm(dpre, wd_ref) + bd_ref[...]
    delta_ref[...] = jnp.maximum(z, 0.0) + jnp.log1p(jnp.exp(-jnp.abs(z)))

    @pl.when(i == 0)
    def _():
        ucs_ref[...] = jnp.zeros_like(ucs_ref)
    ucs_ref[0:1, :] += jnp.sum(x_gnn * cc, axis=0, keepdims=True)
    ucs_ref[1:2, :] += jnp.sum(cc, axis=0, keepdims=True)


def _dense(x, a0, a1, cd0, cd1, cs0, cs1, Wl, bl, Wr, Wr1, br1, Wr2, br2,
           Wdel, Wc, Wres, Wd, bd):
    grid = N // BN
    row = lambda i: (i, 0)
    full = lambda i: (0, 0)
    rspec = pl.BlockSpec((BN, D), row)
    cspec = pl.BlockSpec((BN, 1), row)
    return pl.pallas_call(
        _dense_body,
        grid=(grid,),
        in_specs=[rspec, rspec, rspec, cspec, cspec, cspec, cspec,
                  pl.BlockSpec((D, D), full), pl.BlockSpec((1, D), full),
                  pl.BlockSpec((D, D), full),
                  pl.BlockSpec((32, D), full), pl.BlockSpec((1, 32), full),
                  pl.BlockSpec((1, 32), full), pl.BlockSpec((1, 1), full),
                  pl.BlockSpec((D, D), full), pl.BlockSpec((D, D), full),
                  pl.BlockSpec((D, D), full), pl.BlockSpec((D, D), full),
                  pl.BlockSpec((1, D), full)],
        out_specs=[cspec, rspec, rspec, pl.BlockSpec((8, D), full)],
        out_shape=[jax.ShapeDtypeStruct((N, 1), jnp.float32),
                   jax.ShapeDtypeStruct((N, D), jnp.float32),
                   jax.ShapeDtypeStruct((N, D), jnp.float32),
                   jax.ShapeDtypeStruct((8, D), jnp.float32)],
        compiler_params=pltpu.CompilerParams(
            dimension_semantics=("arbitrary",)),
    )(x, a0, a1, cd0, cd1, cs0, cs1, Wl, bl, Wr, Wr1, br1, Wr2, br2,
      Wdel, Wc, Wres, Wd, bd)


RB = 128
RC = 128


def _rank_body(si_ref, srow_ref, rank_ref):
    ib = pl.program_id(0)
    si = jnp.broadcast_to(si_ref[...], (RB, RC))

    def pre(k, acc):
        sj = jnp.broadcast_to(srow_ref[0:1, pl.ds(k * RC, RC)], (RB, RC))
        return acc + (sj >= si).astype(jnp.int32)

    def post(k, acc):
        sj = jnp.broadcast_to(srow_ref[0:1, pl.ds(k * RC, RC)], (RB, RC))
        return acc + (sj > si).astype(jnp.int32)

    acc = lax.fori_loop(0, ib, pre, jnp.zeros((RB, RC), jnp.int32))
    acc = lax.fori_loop(ib + 1, NP // RC, post, acc)
    sj = jnp.broadcast_to(srow_ref[0:1, pl.ds(ib * RC, RC)], (RB, RC))
    gi = lax.broadcasted_iota(jnp.int32, (RB, RC), 0)
    gj = lax.broadcasted_iota(jnp.int32, (RB, RC), 1)
    cmp = (sj > si) | ((sj == si) & (gj < gi))
    acc = acc + cmp.astype(jnp.int32)
    rank_ref[...] = jnp.sum(acc, axis=1, keepdims=True)


def _rank(s_col, s_row):
    return pl.pallas_call(
        _rank_body,
        grid=(NP // RB,),
        in_specs=[pl.BlockSpec((RB, 1), lambda i: (i, 0)),
                  pl.BlockSpec((1, NP), lambda i: (0, 0))],
        out_specs=pl.BlockSpec((RB, 1), lambda i: (i, 0)),
        out_shape=jax.ShapeDtypeStruct((NP, 1), jnp.int32),
        compiler_params=pltpu.CompilerParams(
            dimension_semantics=("arbitrary",)),
    )(s_col, s_row)


RPW = NP // NW
RCH = 80


def _permute_rows(rows, rank, scatter: bool):
    mesh = plsc.VectorSubcoreMesh(core_axis_name="c", subcore_axis_name="s")

    @functools.partial(
        pl.kernel,
        out_type=jax.ShapeDtypeStruct((NP, D), jnp.float32),
        mesh=mesh,
        scratch_types=[
            pltpu.VMEM((RCH,), jnp.int32),
            pltpu.VMEM((RCH, D), jnp.float32),
            pltpu.SemaphoreType.DMA,
        ],
        compiler_params=pltpu.CompilerParams(needs_layout_passes=False),
    )
    def k(rows_hbm, rank_hbm, out_hbm, idx_v, buf_v, sem):
        c = lax.axis_index("c")
        sid = lax.axis_index("s")
        base = (sid * 2 + c) * RPW

        def body(e, _):
            off = base + e * RCH
            pltpu.sync_copy(rank_hbm.at[pl.ds(off, RCH)], idx_v)
            if scatter:
                pltpu.sync_copy(rows_hbm.at[pl.ds(off, RCH), :], buf_v)
                pltpu.async_copy(buf_v, out_hbm.at[idx_v], sem).wait()
            else:
                pltpu.async_copy(rows_hbm.at[idx_v], buf_v, sem).wait()
                pltpu.sync_copy(buf_v, out_hbm.at[pl.ds(off, RCH), :])
            return 0

        lax.fori_loop(0, RPW // RCH, body, 0)

    return k(rows, rank)


CB = 256


def _cumsum_body(x_ref, o_ref, carry_ref):
    i = pl.program_id(0)

    @pl.when(i == 0)
    def _():
        carry_ref[...] = jnp.zeros_like(carry_ref)

    blk = x_ref[...]
    ri = lax.broadcasted_iota(jnp.int32, (CB, CB), 0)
    ci = lax.broadcasted_iota(jnp.int32, (CB, CB), 1)
    L = (ri >= ci).astype(jnp.float32)
    c = carry_ref[0:1, :]
    o_ref[...] = lax.dot_general(L, blk, (((1,), (0,)), ((), ())),
                                 preferred_element_type=jnp.float32) + c
    carry_ref[0:1, :] = c + jnp.sum(blk, axis=0, keepdims=True)


def _cumsum(xs):
    return pl.pallas_call(
        _cumsum_body,
        grid=(NP // CB,),
        in_specs=[pl.BlockSpec((CB, D), lambda i: (i, 0))],
        out_specs=pl.BlockSpec((CB, D), lambda i: (i, 0)),
        out_shape=jax.ShapeDtypeStruct((NP, D), jnp.float32),
        scratch_shapes=[pltpu.VMEM((8, D), jnp.float32)],
        compiler_params=pltpu.CompilerParams(
            dimension_semantics=("arbitrary",)),
    )(xs)


def _final_body(x_ref, d_ref, p_ref, r_ref, ucs_ref, a_ref, bp_ref, dp_ref,
                o_ref):
    a = a_ref[...]
    bp = bp_ref[...]
    c1 = bp * ucs_ref[0:1, :]
    c2 = bp * ucs_ref[1:2, :]
    dp = dp_ref[0, 0]
    y = jnp.exp(d_ref[...] * a) * c1 + p_ref[...] * c2
    o = y + r_ref[...] * dp
    mu = jnp.mean(o, axis=1, keepdims=True)
    dev = o - mu
    var = jnp.mean(dev * dev, axis=1, keepdims=True)
    o_ref[...] = x_ref[...] + dev * lax.rsqrt(var + 1e-5)


def _final(x, delta, P, res, ucs, a, bp, dp):
    row = lambda i: (i, 0)
    full = lambda i: (0, 0)
    rspec = pl.BlockSpec((BN, D), row)
    return pl.pallas_call(
        _final_body,
        grid=(N // BN,),
        in_specs=[rspec, rspec, rspec, rspec,
                  pl.BlockSpec((8, D), full), pl.BlockSpec((1, D), full),
                  pl.BlockSpec((1, D), full), pl.BlockSpec((1, 1), full)],
        out_specs=rspec,
        out_shape=jax.ShapeDtypeStruct((N, D), jnp.float32),
        compiler_params=pltpu.CompilerParams(
            dimension_semantics=("arbitrary",)),
    )(x, delta, P, res, ucs, a, bp, dp)


def kernel(x, edge_index, Wl, bl, Wr, Wr1, br1, Wr2, br2, Wproj, A, Bp, Dp,
           Wd, bd):
    src = edge_index[0]
    dst = edge_index[1]

    aggp, hist = _edge_agg(x, src, dst)

    scores, delta, res, ucs = _dense(
        x, aggp[0, :N], aggp[1, :N],
        hist[0, 0, :, None], hist[1, 0, :, None],
        hist[0, 1, :, None], hist[1, 1, :, None],
        Wl, bl.reshape(1, D), Wr, Wr1, br1.reshape(1, 32),
        Wr2, br2.reshape(1, 1),
        Wproj[:D], Wproj[2 * D:3 * D], Wproj[3 * D:], Wd, bd.reshape(1, D))

    s_pad = jnp.concatenate(
        [scores, jnp.full((NP - N, 1), -jnp.inf, jnp.float32)], axis=0)
    rank = _rank(s_pad, s_pad.reshape(1, NP))
    rank_flat = rank.reshape(NP)

    delta_pad = jnp.concatenate(
        [delta, jnp.zeros((NP - N, D), jnp.float32)], axis=0)
    ds = _permute_rows(delta_pad, rank_flat, scatter=True)
    Ps = _cumsum(ds)
    Ppad = _permute_rows(Ps, rank_flat, scatter=False)

    return _final(x, delta, Ppad[:N], res, ucs,
                  A.reshape(1, D), Bp.reshape(1, D), Dp.reshape(1, 1))


# speedup vs baseline: 1.0922x; 1.0922x over previous
import functools

import jax
import jax.numpy as jnp
from jax import lax
from jax.experimental import pallas as pl
from jax.experimental.pallas import tpu as pltpu
from jax.experimental.pallas import tpu_sc as plsc

N = 10000
E = 320000
D = 128
NP = 10240
NW = 32
EPW = E // NW
EC = 80
NCHUNK = EPW // EC
ROWS_PER_TILE = NP // 16
ZB = 128


def _edge_agg(x, src, dst):
    mesh = plsc.VectorSubcoreMesh(core_axis_name="c", subcore_axis_name="s")

    @functools.partial(
        pl.kernel,
        out_type=[
            jax.ShapeDtypeStruct((2, NP, D), jnp.float32),
            jax.ShapeDtypeStruct((2, 2, N), jnp.float32),
        ],
        mesh=mesh,
        scratch_types=[
            pltpu.VMEM((EC,), jnp.int32),
            pltpu.VMEM((EC,), jnp.int32),
            pltpu.VMEM((EC,), jnp.int32),
            pltpu.VMEM((EC,), jnp.int32),
            pltpu.VMEM((EC,), jnp.int32),
            pltpu.VMEM((EC,), jnp.int32),
            pltpu.VMEM((EC, D), jnp.float32),
            pltpu.VMEM((EC, D), jnp.float32),
            pltpu.VMEM((N,), jnp.float32),
            pltpu.VMEM((N,), jnp.float32),
            pltpu.VMEM_SHARED((NP, D), jnp.float32),
            pltpu.VMEM_SHARED((N,), jnp.float32),
            pltpu.VMEM_SHARED((N,), jnp.float32),
            pltpu.SemaphoreType.DMA,
            pltpu.SemaphoreType.DMA,
            pltpu.SemaphoreType.DMA,
            pltpu.SemaphoreType.DMA,
            pltpu.SemaphoreType.DMA,
            pltpu.SemaphoreType.DMA,
        ],
        compiler_params=pltpu.CompilerParams(needs_layout_passes=False),
    )
    def k(x_hbm, src_hbm, dst_hbm, agg_out, hist_out,
          srcA, dstA, srcB, dstB, dstSA, dstSB, rows_a, rows_b, hd_loc, hs_loc,
          agg_sh, hd_sh, hs_sh, sia, sib, sra, srb, ssa, ssb):
        c = lax.axis_index("c")
        sid = lax.axis_index("s")
        z16 = jnp.zeros((16,), jnp.float32)
        ones16 = jnp.ones((16,), jnp.float32)
        wid = sid * 2 + c

        def start_idx(kk, sv, dv, sem):
            off = pl.multiple_of(wid * EPW + kk * EC, 8)
            pltpu.async_copy(src_hbm.at[pl.ds(off, EC)], sv, sem)
            pltpu.async_copy(dst_hbm.at[pl.ds(off, EC)], dv, sem)

        def wait_idx(sv, dv, sem):
            pltpu.make_async_copy(src_hbm.at[pl.ds(0, EC)], sv, sem).wait()
            pltpu.make_async_copy(dst_hbm.at[pl.ds(0, EC)], dv, sem).wait()

        def start_gather(sv, buf, sem):
            pltpu.async_copy(x_hbm.at[sv], buf, sem)

        def wait_gather(buf, sem):
            pltpu.make_async_copy(x_hbm.at[srcA], buf, sem).wait()

        def hists_and_stage(sv, dv, ds_buf):
            for j in range(EC // 16):
                di = dv[pl.ds(j * 16, 16)]
                si = sv[pl.ds(j * 16, 16)]
                ds_buf[pl.ds(j * 16, 16)] = di
                plsc.addupdate_scatter(hd_loc, [di], ones16)
                plsc.addupdate_scatter(hs_loc, [si], ones16)

        def wait_scatter(rows, ds_buf, sem):
            pltpu.make_async_copy(rows, agg_sh.at[ds_buf], sem).wait()

        start_idx(0, srcA, dstA, sia)
        start_idx(1, srcB, dstB, sib)

        def zloop(i, _):
            hd_loc[pl.ds(i * 16, 16)] = z16
            hs_loc[pl.ds(i * 16, 16)] = z16
            return 0
        lax.fori_loop(0, N // 16, zloop, 0)

        def zloop2(i, _):
            for j in range(D // 16):
                rows_a[i, pl.ds(j * 16, 16)] = z16
            return 0
        lax.fori_loop(0, EC, zloop2, 0)

        r0 = sid * ROWS_PER_TILE
        for t in range(ROWS_PER_TILE // EC):
            pltpu.sync_copy(rows_a, agg_sh.at[pl.ds(r0 + t * EC, EC), :])
        @pl.when(sid == 0)
        def _():
            pltpu.sync_copy(hd_loc, hd_sh)
            pltpu.sync_copy(hs_loc, hs_sh)

        plsc.subcore_barrier()

        wait_idx(srcA, dstA, sia)
        start_gather(srcA, rows_a, sra)
        wait_idx(srcB, dstB, sib)
        start_gather(srcB, rows_b, srb)

        def half(kk, sv, dv, dsb, rows, si_, sr_, ss_):
            wait_gather(rows, sr_)
            hists_and_stage(sv, dv, dsb)
            pltpu.async_copy(rows, agg_sh.at[dsb], ss_, add=True)
            @pl.when(kk + 2 < NCHUNK)
            def _():
                start_idx(kk + 2, sv, dv, si_)
            wait_scatter(rows, dsb, ss_)
            @pl.when(kk + 2 < NCHUNK)
            def _():
                wait_idx(sv, dv, si_)
                start_gather(sv, rows, sr_)

        def body(i, _):
            ka = 2 * i
            half(ka, srcA, dstA, dstSA, rows_a, sia, sra, ssa)
            half(ka + 1, srcB, dstB, dstSB, rows_b, sib, srb, ssb)
            return 0

        lax.fori_loop(0, (NCHUNK - 1) // 2, body, 0)
        half(NCHUNK - 1, srcA, dstA, dstSA, rows_a, sia, sra, ssa)
        plsc.subcore_barrier()

        def merge(e, _):
            off = e * EC
            for j in range(EC // 16):
                srcA[pl.ds(j * 16, 16)] = off + j * 16 + lax.iota(jnp.int32, 16)
            pltpu.sync_copy(hd_loc.at[pl.ds(off, EC)], hd_sh.at[srcA], add=True)
            pltpu.sync_copy(hs_loc.at[pl.ds(off, EC)], hs_sh.at[srcA], add=True)
            return 0

        lax.fori_loop(0, N // EC, merge, 0)
        plsc.subcore_barrier()

        pltpu.sync_copy(agg_sh.at[pl.ds(r0, ROWS_PER_TILE), :],
                        agg_out.at[c, pl.ds(r0, ROWS_PER_TILE), :])
        @pl.when(sid == 0)
        def _():
            pltpu.sync_copy(hd_sh, hist_out.at[c, 0, :])
            pltpu.sync_copy(hs_sh, hist_out.at[c, 1, :])

    return k(x, src, dst)


BN = 1000


def _dense_body(x_ref, a0_ref, a1_ref, cd0_ref, cd1_ref, cs0_ref, cs1_ref,
                wl_ref, bl_ref, wr_ref, wr1_ref, br1_ref, wr2_ref, br2_ref,
                wdel_ref, wc_ref, wres_ref, wd_ref, bd_ref,
                scores_ref, delta_ref, res_ref, ucs_ref):
    i = pl.program_id(0)
    x = x_ref[...]
    cnt = cd0_ref[...] + cd1_ref[...]
    deg = cnt + cs0_ref[...] + cs1_ref[...]
    mean = (a0_ref[...] + a1_ref[...]) / jnp.maximum(cnt, 1.0)

    def mm(a, w_ref):
        return lax.dot_general(a, w_ref[...], (((1,), (1,)), ((), ())),
                               preferred_element_type=jnp.float32)

    pre = mm(mean, wl_ref) + bl_ref[...] + mm(x, wr_ref) + x
    x_gnn = 0.5 * pre * (1.0 + lax.erf(pre * 0.7071067811865476))
    h1 = jnp.maximum(mm(x_gnn, wr1_ref) + br1_ref[...], 0.0)
    sc = jnp.sum(h1 * wr2_ref[...], axis=1, keepdims=True) + br2_ref[0, 0]
    scores_ref[...] = sc + deg

    dpre = mm(x_gnn, wdel_ref)
    cc = mm(x_gnn, wc_ref)
    res_ref[...] = mm(x_gnn, wres_ref)
    z = mm(dpre, wd_ref) + bd_ref[...]
    delta_ref[...] = jnp.maximum(z, 0.0) + jnp.log1p(jnp.exp(-jnp.abs(z)))

    @pl.when(i == 0)
    def _():
        ucs_ref[...] = jnp.zeros_like(ucs_ref)
    ucs_ref[0:1, :] += jnp.sum(x_gnn * cc, axis=0, keepdims=True)
    ucs_ref[1:2, :] += jnp.sum(cc, axis=0, keepdims=True)


def _dense(x, a0, a1, cd0, cd1, cs0, cs1, Wl, bl, Wr, Wr1, br1, Wr2, br2,
           Wdel, Wc, Wres, Wd, bd):
    grid = N // BN
    row = lambda i: (i, 0)
    full = lambda i: (0, 0)
    rspec = pl.BlockSpec((BN, D), row)
    cspec = pl.BlockSpec((BN, 1), row)
    return pl.pallas_call(
        _dense_body,
        grid=(grid,),
        in_specs=[rspec, rspec, rspec, cspec, cspec, cspec, cspec,
                  pl.BlockSpec((D, D), full), pl.BlockSpec((1, D), full),
                  pl.BlockSpec((D, D), full),
                  pl.BlockSpec((32, D), full), pl.BlockSpec((1, 32), full),
                  pl.BlockSpec((1, 32), full), pl.BlockSpec((1, 1), full),
                  pl.BlockSpec((D, D), full), pl.BlockSpec((D, D), full),
                  pl.BlockSpec((D, D), full), pl.BlockSpec((D, D), full),
                  pl.BlockSpec((1, D), full)],
        out_specs=[cspec, rspec, rspec, pl.BlockSpec((8, D), full)],
        out_shape=[jax.ShapeDtypeStruct((N, 1), jnp.float32),
                   jax.ShapeDtypeStruct((N, D), jnp.float32),
                   jax.ShapeDtypeStruct((N, D), jnp.float32),
                   jax.ShapeDtypeStruct((8, D), jnp.float32)],
        compiler_params=pltpu.CompilerParams(
            dimension_semantics=("arbitrary",)),
    )(x, a0, a1, cd0, cd1, cs0, cs1, Wl, bl, Wr, Wr1, br1, Wr2, br2,
      Wdel, Wc, Wres, Wd, bd)


RB = 128
RC = 128


def _rank_body(si_ref, srow_ref, rank_ref):
    ib = pl.program_id(0)
    si = jnp.broadcast_to(si_ref[...], (RB, RC))

    one = jnp.float32(1.0)
    zero = jnp.float32(0.0)

    def pre(k, acc):
        sj = jnp.broadcast_to(srow_ref[0:1, pl.ds(k * RC, RC)], (RB, RC))
        return acc + jnp.where(sj >= si, one, zero)

    def post(k, acc):
        sj = jnp.broadcast_to(srow_ref[0:1, pl.ds(k * RC, RC)], (RB, RC))
        return acc + jnp.where(sj > si, one, zero)

    acc = lax.fori_loop(0, ib, pre, jnp.zeros((RB, RC), jnp.float32))
    acc = lax.fori_loop(ib + 1, NP // RC, post, acc)
    sj = jnp.broadcast_to(srow_ref[0:1, pl.ds(ib * RC, RC)], (RB, RC))
    gi = lax.broadcasted_iota(jnp.int32, (RB, RC), 0)
    gj = lax.broadcasted_iota(jnp.int32, (RB, RC), 1)
    cmp = (sj > si) | ((sj == si) & (gj < gi))
    acc = acc + jnp.where(cmp, one, zero)
    rank_ref[...] = jnp.sum(acc, axis=1, keepdims=True).astype(jnp.int32)


def _rank(s_col, s_row):
    return pl.pallas_call(
        _rank_body,
        grid=(NP // RB,),
        in_specs=[pl.BlockSpec((RB, 1), lambda i: (i, 0)),
                  pl.BlockSpec((1, NP), lambda i: (0, 0))],
        out_specs=pl.BlockSpec((RB, 1), lambda i: (i, 0)),
        out_shape=jax.ShapeDtypeStruct((NP, 1), jnp.int32),
        compiler_params=pltpu.CompilerParams(
            dimension_semantics=("arbitrary",)),
    )(s_col, s_row)


RPW = NP // NW
RCH = 80


def _permute_rows(rows, rank, scatter: bool):
    mesh = plsc.VectorSubcoreMesh(core_axis_name="c", subcore_axis_name="s")

    @functools.partial(
        pl.kernel,
        out_type=jax.ShapeDtypeStruct((NP, D), jnp.float32),
        mesh=mesh,
        scratch_types=[
            pltpu.VMEM((RCH,), jnp.int32),
            pltpu.VMEM((RCH, D), jnp.float32),
            pltpu.SemaphoreType.DMA,
        ],
        compiler_params=pltpu.CompilerParams(needs_layout_passes=False),
    )
    def k(rows_hbm, rank_hbm, out_hbm, idx_v, buf_v, sem):
        c = lax.axis_index("c")
        sid = lax.axis_index("s")
        base = (sid * 2 + c) * RPW

        def body(e, _):
            off = base + e * RCH
            pltpu.sync_copy(rank_hbm.at[pl.ds(off, RCH)], idx_v)
            if scatter:
                pltpu.sync_copy(rows_hbm.at[pl.ds(off, RCH), :], buf_v)
                pltpu.async_copy(buf_v, out_hbm.at[idx_v], sem).wait()
            else:
                pltpu.async_copy(rows_hbm.at[idx_v], buf_v, sem).wait()
                pltpu.sync_copy(buf_v, out_hbm.at[pl.ds(off, RCH), :])
            return 0

        lax.fori_loop(0, RPW // RCH, body, 0)

    return k(rows, rank)


CB = 256


def _cumsum_body(x_ref, o_ref, carry_ref):
    i = pl.program_id(0)

    @pl.when(i == 0)
    def _():
        carry_ref[...] = jnp.zeros_like(carry_ref)

    blk = x_ref[...]
    ri = lax.broadcasted_iota(jnp.int32, (CB, CB), 0)
    ci = lax.broadcasted_iota(jnp.int32, (CB, CB), 1)
    L = (ri >= ci).astype(jnp.float32)
    c = carry_ref[0:1, :]
    o_ref[...] = lax.dot_general(L, blk, (((1,), (0,)), ((), ())),
                                 preferred_element_type=jnp.float32) + c
    carry_ref[0:1, :] = c + jnp.sum(blk, axis=0, keepdims=True)


def _cumsum(xs):
    return pl.pallas_call(
        _cumsum_body,
        grid=(NP // CB,),
        in_specs=[pl.BlockSpec((CB, D), lambda i: (i, 0))],
        out_specs=pl.BlockSpec((CB, D), lambda i: (i, 0)),
        out_shape=jax.ShapeDtypeStruct((NP, D), jnp.float32),
        scratch_shapes=[pltpu.VMEM((8, D), jnp.float32)],
        compiler_params=pltpu.CompilerParams(
            dimension_semantics=("arbitrary",)),
    )(xs)


def _final_body(x_ref, d_ref, p_ref, r_ref, ucs_ref, a_ref, bp_ref, dp_ref,
                o_ref):
    a = a_ref[...]
    bp = bp_ref[...]
    c1 = bp * ucs_ref[0:1, :]
    c2 = bp * ucs_ref[1:2, :]
    dp = dp_ref[0, 0]
    y = jnp.exp(d_ref[...] * a) * c1 + p_ref[...] * c2
    o = y + r_ref[...] * dp
    mu = jnp.mean(o, axis=1, keepdims=True)
    dev = o - mu
    var = jnp.mean(dev * dev, axis=1, keepdims=True)
    o_ref[...] = x_ref[...] + dev * lax.rsqrt(var + 1e-5)


def _final(x, delta, P, res, ucs, a, bp, dp):
    row = lambda i: (i, 0)
    full = lambda i: (0, 0)
    rspec = pl.BlockSpec((BN, D), row)
    return pl.pallas_call(
        _final_body,
        grid=(N // BN,),
        in_specs=[rspec, rspec, rspec, rspec,
                  pl.BlockSpec((8, D), full), pl.BlockSpec((1, D), full),
                  pl.BlockSpec((1, D), full), pl.BlockSpec((1, 1), full)],
        out_specs=rspec,
        out_shape=jax.ShapeDtypeStruct((N, D), jnp.float32),
        compiler_params=pltpu.CompilerParams(
            dimension_semantics=("arbitrary",)),
    )(x, delta, P, res, ucs, a, bp, dp)


def kernel(x, edge_index, Wl, bl, Wr, Wr1, br1, Wr2, br2, Wproj, A, Bp, Dp,
           Wd, bd):
    src = edge_index[0]
    dst = edge_index[1]

    aggp, hist = _edge_agg(x, src, dst)

    scores, delta, res, ucs = _dense(
        x, aggp[0, :N], aggp[1, :N],
        hist[0, 0, :, None], hist[1, 0, :, None],
        hist[0, 1, :, None], hist[1, 1, :, None],
        Wl, bl.reshape(1, D), Wr, Wr1, br1.reshape(1, 32),
        Wr2, br2.reshape(1, 1),
        Wproj[:D], Wproj[2 * D:3 * D], Wproj[3 * D:], Wd, bd.reshape(1, D))

    s_pad = jnp.concatenate(
        [scores, jnp.full((NP - N, 1), -jnp.inf, jnp.float32)], axis=0)
    rank = _rank(s_pad, s_pad.reshape(1, NP))
    rank_flat = rank.reshape(NP)

    delta_pad = jnp.concatenate(
        [delta, jnp.zeros((NP - N, D), jnp.float32)], axis=0)
    ds = _permute_rows(delta_pad, rank_flat, scatter=True)
    Ps = _cumsum(ds)
    Ppad = _permute_rows(Ps, rank_flat, scatter=False)

    return _final(x, delta, Ppad[:N], res, ucs,
                  A.reshape(1, D), Bp.reshape(1, D), Dp.reshape(1, 1))


# speedup vs baseline: 1.1586x; 1.0608x over previous
import functools

import jax
import jax.numpy as jnp
from jax import lax
from jax.experimental import pallas as pl
from jax.experimental.pallas import tpu as pltpu
from jax.experimental.pallas import tpu_sc as plsc

N = 10000
E = 320000
D = 128
NP = 10240
NW = 32
EPW = E // NW
EC = 80
NCHUNK = EPW // EC
ROWS_PER_TILE = NP // 16
ZB = 128


def _edge_agg(x, src, dst):
    mesh = plsc.VectorSubcoreMesh(core_axis_name="c", subcore_axis_name="s")

    @functools.partial(
        pl.kernel,
        out_type=[
            jax.ShapeDtypeStruct((2, NP, D), jnp.float32),
            jax.ShapeDtypeStruct((2, 2, N), jnp.float32),
        ],
        mesh=mesh,
        scratch_types=[
            pltpu.VMEM((EC,), jnp.int32),
            pltpu.VMEM((EC,), jnp.int32),
            pltpu.VMEM((EC,), jnp.int32),
            pltpu.VMEM((EC,), jnp.int32),
            pltpu.VMEM((EC,), jnp.int32),
            pltpu.VMEM((EC,), jnp.int32),
            pltpu.VMEM((EC, D), jnp.float32),
            pltpu.VMEM((EC, D), jnp.float32),
            pltpu.VMEM((N,), jnp.float32),
            pltpu.VMEM((N,), jnp.float32),
            pltpu.VMEM_SHARED((NP, D), jnp.float32),
            pltpu.VMEM_SHARED((N,), jnp.float32),
            pltpu.VMEM_SHARED((N,), jnp.float32),
            pltpu.SemaphoreType.DMA,
            pltpu.SemaphoreType.DMA,
            pltpu.SemaphoreType.DMA,
            pltpu.SemaphoreType.DMA,
            pltpu.SemaphoreType.DMA,
            pltpu.SemaphoreType.DMA,
        ],
        compiler_params=pltpu.CompilerParams(needs_layout_passes=False),
    )
    def k(x_hbm, src_hbm, dst_hbm, agg_out, hist_out,
          srcA, dstA, srcB, dstB, dstSA, dstSB, rows_a, rows_b, hd_loc, hs_loc,
          agg_sh, hd_sh, hs_sh, sia, sib, sra, srb, ssa, ssb):
        c = lax.axis_index("c")
        sid = lax.axis_index("s")
        z16 = jnp.zeros((16,), jnp.float32)
        ones16 = jnp.ones((16,), jnp.float32)
        wid = sid * 2 + c

        def start_idx(kk, sv, dv, sem):
            off = pl.multiple_of(wid * EPW + kk * EC, 8)
            pltpu.async_copy(src_hbm.at[pl.ds(off, EC)], sv, sem)
            pltpu.async_copy(dst_hbm.at[pl.ds(off, EC)], dv, sem)

        def wait_idx(sv, dv, sem):
            pltpu.make_async_copy(src_hbm.at[pl.ds(0, EC)], sv, sem).wait()
            pltpu.make_async_copy(dst_hbm.at[pl.ds(0, EC)], dv, sem).wait()

        def start_gather(sv, buf, sem):
            pltpu.async_copy(x_hbm.at[sv], buf, sem)

        def wait_gather(buf, sem):
            pltpu.make_async_copy(x_hbm.at[srcA], buf, sem).wait()

        def hists_and_stage(sv, dv, ds_buf):
            for j in range(EC // 16):
                di = dv[pl.ds(j * 16, 16)]
                si = sv[pl.ds(j * 16, 16)]
                ds_buf[pl.ds(j * 16, 16)] = di
                plsc.addupdate_scatter(hd_loc, [di], ones16)
                plsc.addupdate_scatter(hs_loc, [si], ones16)

        def wait_scatter(rows, ds_buf, sem):
            pltpu.make_async_copy(rows, agg_sh.at[ds_buf], sem).wait()

        start_idx(0, srcA, dstA, sia)
        start_idx(1, srcB, dstB, sib)

        def zloop(i, _):
            hd_loc[pl.ds(i * 16, 16)] = z16
            hs_loc[pl.ds(i * 16, 16)] = z16
            return 0
        lax.fori_loop(0, N // 16, zloop, 0)

        def zloop2(i, _):
            for j in range(D // 16):
                rows_a[i, pl.ds(j * 16, 16)] = z16
            return 0
        lax.fori_loop(0, EC, zloop2, 0)

        r0 = sid * ROWS_PER_TILE
        for t in range(ROWS_PER_TILE // EC):
            pltpu.sync_copy(rows_a, agg_sh.at[pl.ds(r0 + t * EC, EC), :])
        @pl.when(sid == 0)
        def _():
            pltpu.sync_copy(hd_loc, hd_sh)
            pltpu.sync_copy(hs_loc, hs_sh)

        plsc.subcore_barrier()

        wait_idx(srcA, dstA, sia)
        start_gather(srcA, rows_a, sra)
        wait_idx(srcB, dstB, sib)
        start_gather(srcB, rows_b, srb)

        def half(kk, sv, dv, dsb, rows, si_, sr_, ss_):
            wait_gather(rows, sr_)
            hists_and_stage(sv, dv, dsb)
            pltpu.async_copy(rows, agg_sh.at[dsb], ss_, add=True)
            @pl.when(kk + 2 < NCHUNK)
            def _():
                start_idx(kk + 2, sv, dv, si_)
            wait_scatter(rows, dsb, ss_)
            @pl.when(kk + 2 < NCHUNK)
            def _():
                wait_idx(sv, dv, si_)
                start_gather(sv, rows, sr_)

        def body(i, _):
            ka = 2 * i
            half(ka, srcA, dstA, dstSA, rows_a, sia, sra, ssa)
            half(ka + 1, srcB, dstB, dstSB, rows_b, sib, srb, ssb)
            return 0

        lax.fori_loop(0, (NCHUNK - 1) // 2, body, 0)
        half(NCHUNK - 1, srcA, dstA, dstSA, rows_a, sia, sra, ssa)
        plsc.subcore_barrier()

        def merge(e, _):
            off = e * EC
            for j in range(EC // 16):
                srcA[pl.ds(j * 16, 16)] = off + j * 16 + lax.iota(jnp.int32, 16)
            pltpu.sync_copy(hd_loc.at[pl.ds(off, EC)], hd_sh.at[srcA], add=True)
            pltpu.sync_copy(hs_loc.at[pl.ds(off, EC)], hs_sh.at[srcA], add=True)
            return 0

        lax.fori_loop(0, N // EC, merge, 0)
        plsc.subcore_barrier()

        pltpu.sync_copy(agg_sh.at[pl.ds(r0, ROWS_PER_TILE), :],
                        agg_out.at[c, pl.ds(r0, ROWS_PER_TILE), :])
        @pl.when(sid == 0)
        def _():
            pltpu.sync_copy(hd_sh, hist_out.at[c, 0, :])
            pltpu.sync_copy(hs_sh, hist_out.at[c, 1, :])

    return k(x, src, dst)


BN = 1000


def _dense_body(x_ref, a0_ref, a1_ref, cd0_ref, cd1_ref, cs0_ref, cs1_ref,
                si_ref,
                wl_ref, bl_ref, wr_ref, wr1_ref, br1_ref, wr2_ref, br2_ref,
                wdel_ref, wc_ref, wres_ref, wd_ref, bd_ref,
                scores_ref, delta_ref, res_ref, ucs_ref):
    i = pl.program_id(0)
    x = x_ref[...]
    cnt = cd0_ref[...] + cd1_ref[...]
    deg = cnt + cs0_ref[...] + cs1_ref[...]
    mean = (a0_ref[0] + a1_ref[0]) / jnp.maximum(cnt, 1.0)

    def mm(a, w_ref):
        return lax.dot_general(a, w_ref[...], (((1,), (1,)), ((), ())),
                               preferred_element_type=jnp.float32)

    pre = mm(mean, wl_ref) + bl_ref[...] + mm(x, wr_ref) + x
    x_gnn = 0.5 * pre * (1.0 + lax.erf(pre * 0.7071067811865476))
    h1 = jnp.maximum(mm(x_gnn, wr1_ref) + br1_ref[...], 0.0)
    sc = jnp.sum(h1 * wr2_ref[...], axis=1, keepdims=True) + br2_ref[0, 0]
    scores_ref[...] = sc + deg

    dpre = mm(x_gnn, wdel_ref)
    cc = mm(x_gnn, wc_ref)
    res_ref[...] = mm(x_gnn, wres_ref)
    z = mm(dpre, wd_ref) + bd_ref[...]
    delta_ref[...] = jnp.maximum(z, 0.0) + jnp.log1p(jnp.exp(-jnp.abs(z)))

    @pl.when(i == 0)
    def _():
        ucs_ref[...] = jnp.zeros_like(ucs_ref)
    ucs_ref[0:1, :] += jnp.sum(x_gnn * cc, axis=0, keepdims=True)
    ucs_ref[1:2, :] += jnp.sum(cc, axis=0, keepdims=True)


def _dense(x, aggp, hist, sinit, Wl, bl, Wr, Wr1, br1, Wr2, br2,
           Wdel, Wc, Wres, Wd, bd):
    grid = N // BN
    row = lambda i: (i, 0)
    full = lambda i: (0, 0)
    rspec = pl.BlockSpec((BN, D), row)
    aspec0 = pl.BlockSpec((1, BN, D), lambda i: (0, i, 0))
    aspec1 = pl.BlockSpec((1, BN, D), lambda i: (1, i, 0))
    cspec = pl.BlockSpec((BN, 1), row)
    return pl.pallas_call(
        _dense_body,
        grid=(grid,),
        in_specs=[rspec, aspec0, aspec1,
                  cspec, cspec, cspec, cspec,
                  pl.BlockSpec((BN, 1), row),
                  pl.BlockSpec((D, D), full), pl.BlockSpec((1, D), full),
                  pl.BlockSpec((D, D), full),
                  pl.BlockSpec((32, D), full), pl.BlockSpec((1, 32), full),
                  pl.BlockSpec((1, 32), full), pl.BlockSpec((1, 1), full),
                  pl.BlockSpec((D, D), full), pl.BlockSpec((D, D), full),
                  pl.BlockSpec((D, D), full), pl.BlockSpec((D, D), full),
                  pl.BlockSpec((1, D), full)],
        out_specs=[pl.BlockSpec((BN, 1), row), rspec, rspec,
                   pl.BlockSpec((8, D), full)],
        out_shape=[jax.ShapeDtypeStruct((NP, 1), jnp.float32),
                   jax.ShapeDtypeStruct((NP, D), jnp.float32),
                   jax.ShapeDtypeStruct((NP, D), jnp.float32),
                   jax.ShapeDtypeStruct((8, D), jnp.float32)],
        input_output_aliases={7: 0},
        compiler_params=pltpu.CompilerParams(
            dimension_semantics=("arbitrary",)),
    )(x, aggp, aggp, hist[0, 0, :, None], hist[1, 0, :, None],
      hist[0, 1, :, None], hist[1, 1, :, None], sinit, Wl, bl, Wr, Wr1, br1,
      Wr2, br2, Wdel, Wc, Wres, Wd, bd)
RB = 128
RC = 128


def _rank_body(si_ref, srow_ref, rank_ref):
    ib = pl.program_id(0)
    si = jnp.broadcast_to(si_ref[...], (RB, RC))

    one = jnp.float32(1.0)
    zero = jnp.float32(0.0)

    def pre(k, acc):
        sj = jnp.broadcast_to(srow_ref[0:1, pl.ds(k * RC, RC)], (RB, RC))
        return acc + jnp.where(sj >= si, one, zero)

    def post(k, acc):
        sj = jnp.broadcast_to(srow_ref[0:1, pl.ds(k * RC, RC)], (RB, RC))
        return acc + jnp.where(sj > si, one, zero)

    acc = lax.fori_loop(0, ib, pre, jnp.zeros((RB, RC), jnp.float32))
    acc = lax.fori_loop(ib + 1, NP // RC, post, acc)
    sj = jnp.broadcast_to(srow_ref[0:1, pl.ds(ib * RC, RC)], (RB, RC))
    gi = lax.broadcasted_iota(jnp.int32, (RB, RC), 0)
    gj = lax.broadcasted_iota(jnp.int32, (RB, RC), 1)
    cmp = (sj > si) | ((sj == si) & (gj < gi))
    acc = acc + jnp.where(cmp, one, zero)
    rank_ref[...] = jnp.sum(acc, axis=1, keepdims=True).astype(jnp.int32)


def _rank(s_col, s_row):
    return pl.pallas_call(
        _rank_body,
        grid=(NP // RB,),
        in_specs=[pl.BlockSpec((RB, 1), lambda i: (i, 0)),
                  pl.BlockSpec((1, NP), lambda i: (0, 0))],
        out_specs=pl.BlockSpec((RB, 1), lambda i: (i, 0)),
        out_shape=jax.ShapeDtypeStruct((NP, 1), jnp.int32),
        compiler_params=pltpu.CompilerParams(
            dimension_semantics=("arbitrary",)),
    )(s_col, s_row)


RPW = NP // NW
RCH = 80


def _permute_rows(rows, rank, scatter: bool):
    mesh = plsc.VectorSubcoreMesh(core_axis_name="c", subcore_axis_name="s")

    @functools.partial(
        pl.kernel,
        out_type=jax.ShapeDtypeStruct((NP, D), jnp.float32),
        mesh=mesh,
        scratch_types=[
            pltpu.VMEM((RCH,), jnp.int32),
            pltpu.VMEM((RCH, D), jnp.float32),
            pltpu.SemaphoreType.DMA,
        ],
        compiler_params=pltpu.CompilerParams(needs_layout_passes=False),
    )
    def k(rows_hbm, rank_hbm, out_hbm, idx_v, buf_v, sem):
        c = lax.axis_index("c")
        sid = lax.axis_index("s")
        base = (sid * 2 + c) * RPW

        def body(e, _):
            off = base + e * RCH
            pltpu.sync_copy(rank_hbm.at[pl.ds(off, RCH)], idx_v)
            if scatter:
                pltpu.sync_copy(rows_hbm.at[pl.ds(off, RCH), :], buf_v)
                pltpu.async_copy(buf_v, out_hbm.at[idx_v], sem).wait()
            else:
                pltpu.async_copy(rows_hbm.at[idx_v], buf_v, sem).wait()
                pltpu.sync_copy(buf_v, out_hbm.at[pl.ds(off, RCH), :])
            return 0

        lax.fori_loop(0, RPW // RCH, body, 0)

    return k(rows, rank)


CB = 512


def _cumsum_body(x_ref, o_ref, carry_ref):
    i = pl.program_id(0)

    @pl.when(i == 0)
    def _():
        carry_ref[...] = jnp.zeros_like(carry_ref)

    blk = x_ref[...]
    ri = lax.broadcasted_iota(jnp.int32, (CB, CB), 0)
    ci = lax.broadcasted_iota(jnp.int32, (CB, CB), 1)
    L = (ri >= ci).astype(jnp.float32)
    c = carry_ref[0:1, :]
    o_ref[...] = lax.dot_general(L, blk, (((1,), (0,)), ((), ())),
                                 preferred_element_type=jnp.float32) + c
    carry_ref[0:1, :] = c + jnp.sum(blk, axis=0, keepdims=True)


def _cumsum(xs):
    return pl.pallas_call(
        _cumsum_body,
        grid=(NP // CB,),
        in_specs=[pl.BlockSpec((CB, D), lambda i: (i, 0))],
        out_specs=pl.BlockSpec((CB, D), lambda i: (i, 0)),
        out_shape=jax.ShapeDtypeStruct((NP, D), jnp.float32),
        scratch_shapes=[pltpu.VMEM((8, D), jnp.float32)],
        compiler_params=pltpu.CompilerParams(
            dimension_semantics=("arbitrary",)),
    )(xs)


def _final_body(x_ref, d_ref, p_ref, r_ref, ucs_ref, a_ref, bp_ref, dp_ref,
                o_ref):
    a = a_ref[...]
    bp = bp_ref[...]
    c1 = bp * ucs_ref[0:1, :]
    c2 = bp * ucs_ref[1:2, :]
    dp = dp_ref[0, 0]
    y = jnp.exp(d_ref[...] * a) * c1 + p_ref[...] * c2
    o = y + r_ref[...] * dp
    mu = jnp.mean(o, axis=1, keepdims=True)
    dev = o - mu
    var = jnp.mean(dev * dev, axis=1, keepdims=True)
    o_ref[...] = x_ref[...] + dev * lax.rsqrt(var + 1e-5)


def _final(x, delta, P, res, ucs, a, bp, dp):
    row = lambda i: (i, 0)
    full = lambda i: (0, 0)
    rspec = pl.BlockSpec((BN, D), row)
    return pl.pallas_call(
        _final_body,
        grid=(N // BN,),
        in_specs=[rspec, rspec, rspec, rspec,
                  pl.BlockSpec((8, D), full), pl.BlockSpec((1, D), full),
                  pl.BlockSpec((1, D), full), pl.BlockSpec((1, 1), full)],
        out_specs=rspec,
        out_shape=jax.ShapeDtypeStruct((N, D), jnp.float32),
        compiler_params=pltpu.CompilerParams(
            dimension_semantics=("arbitrary",)),
    )(x, delta, P, res, ucs, a, bp, dp)


def kernel(x, edge_index, Wl, bl, Wr, Wr1, br1, Wr2, br2, Wproj, A, Bp, Dp,
           Wd, bd):
    src = edge_index[0]
    dst = edge_index[1]

    aggp, hist = _edge_agg(x, src, dst)

    sinit = jnp.full((NP, 1), -jnp.inf, jnp.float32)
    scores, delta, res, ucs = _dense(
        x, aggp, hist, sinit,
        Wl, bl.reshape(1, D), Wr, Wr1, br1.reshape(1, 32),
        Wr2, br2.reshape(1, 1),
        Wproj[:D], Wproj[2 * D:3 * D], Wproj[3 * D:], Wd, bd.reshape(1, D))

    rank = _rank(scores, scores.reshape(1, NP))
    rank_flat = rank.reshape(NP)

    ds = _permute_rows(delta, rank_flat, scatter=True)
    Ps = _cumsum(ds)
    Ppad = _permute_rows(Ps, rank_flat, scatter=False)

    return _final(x, delta, Ppad, res, ucs,
                  A.reshape(1, D), Bp.reshape(1, D), Dp.reshape(1, 1))


# speedup vs baseline: 1.1894x; 1.0265x over previous
import functools

import jax
import jax.numpy as jnp
from jax import lax
from jax.experimental import pallas as pl
from jax.experimental.pallas import tpu as pltpu
from jax.experimental.pallas import tpu_sc as plsc

N = 10000
E = 320000
D = 128
NP = 10240
NW = 32
EPW = E // NW
EC = 80
NCHUNK = EPW // EC
ROWS_PER_TILE = NP // 16
ZB = 128


def _edge_agg(x, src, dst):
    mesh = plsc.VectorSubcoreMesh(core_axis_name="c", subcore_axis_name="s")

    @functools.partial(
        pl.kernel,
        out_type=[
            jax.ShapeDtypeStruct((2, NP, D), jnp.float32),
            jax.ShapeDtypeStruct((2, 2, N), jnp.float32),
        ],
        mesh=mesh,
        scratch_types=[
            pltpu.VMEM((EC,), jnp.int32),
            pltpu.VMEM((EC,), jnp.int32),
            pltpu.VMEM((EC,), jnp.int32),
            pltpu.VMEM((EC,), jnp.int32),
            pltpu.VMEM((EC,), jnp.int32),
            pltpu.VMEM((EC,), jnp.int32),
            pltpu.VMEM((EC, D), jnp.float32),
            pltpu.VMEM((EC, D), jnp.float32),
            pltpu.VMEM((N,), jnp.float32),
            pltpu.VMEM((N,), jnp.float32),
            pltpu.VMEM_SHARED((NP, D), jnp.float32),
            pltpu.VMEM_SHARED((N,), jnp.float32),
            pltpu.VMEM_SHARED((N,), jnp.float32),
            pltpu.SemaphoreType.DMA,
            pltpu.SemaphoreType.DMA,
            pltpu.SemaphoreType.DMA,
            pltpu.SemaphoreType.DMA,
            pltpu.SemaphoreType.DMA,
            pltpu.SemaphoreType.DMA,
        ],
        compiler_params=pltpu.CompilerParams(needs_layout_passes=False),
    )
    def k(x_hbm, src_hbm, dst_hbm, agg_out, hist_out,
          srcA, dstA, srcB, dstB, dstSA, dstSB, rows_a, rows_b, hd_loc, hs_loc,
          agg_sh, hd_sh, hs_sh, sia, sib, sra, srb, ssa, ssb):
        c = lax.axis_index("c")
        sid = lax.axis_index("s")
        z16 = jnp.zeros((16,), jnp.float32)
        ones16 = jnp.ones((16,), jnp.float32)
        wid = sid * 2 + c

        def start_idx(kk, sv, dv, sem):
            off = pl.multiple_of(wid * EPW + kk * EC, 8)
            pltpu.async_copy(src_hbm.at[pl.ds(off, EC)], sv, sem)
            pltpu.async_copy(dst_hbm.at[pl.ds(off, EC)], dv, sem)

        def wait_idx(sv, dv, sem):
            pltpu.make_async_copy(src_hbm.at[pl.ds(0, EC)], sv, sem).wait()
            pltpu.make_async_copy(dst_hbm.at[pl.ds(0, EC)], dv, sem).wait()

        def start_gather(sv, buf, sem):
            pltpu.async_copy(x_hbm.at[sv], buf, sem)

        def wait_gather(buf, sem):
            pltpu.make_async_copy(x_hbm.at[srcA], buf, sem).wait()

        def hists_and_stage(sv, dv, ds_buf):
            for j in range(EC // 16):
                di = dv[pl.ds(j * 16, 16)]
                si = sv[pl.ds(j * 16, 16)]
                ds_buf[pl.ds(j * 16, 16)] = di
                plsc.addupdate_scatter(hd_loc, [di], ones16)
                plsc.addupdate_scatter(hs_loc, [si], ones16)

        def wait_scatter(rows, ds_buf, sem):
            pltpu.make_async_copy(rows, agg_sh.at[ds_buf], sem).wait()

        start_idx(0, srcA, dstA, sia)
        start_idx(1, srcB, dstB, sib)

        def zloop(i, _):
            hd_loc[pl.ds(i * 16, 16)] = z16
            hs_loc[pl.ds(i * 16, 16)] = z16
            return 0
        lax.fori_loop(0, N // 16, zloop, 0)

        def zloop2(i, _):
            for j in range(D // 16):
                rows_a[i, pl.ds(j * 16, 16)] = z16
            return 0
        lax.fori_loop(0, EC, zloop2, 0)

        r0 = sid * ROWS_PER_TILE
        for t in range(ROWS_PER_TILE // EC):
            pltpu.sync_copy(rows_a, agg_sh.at[pl.ds(r0 + t * EC, EC), :])
        @pl.when(sid == 0)
        def _():
            pltpu.sync_copy(hd_loc, hd_sh)
            pltpu.sync_copy(hs_loc, hs_sh)

        plsc.subcore_barrier()

        wait_idx(srcA, dstA, sia)
        start_gather(srcA, rows_a, sra)
        wait_idx(srcB, dstB, sib)
        start_gather(srcB, rows_b, srb)

        def half(kk, sv, dv, dsb, rows, si_, sr_, ss_):
            wait_gather(rows, sr_)
            hists_and_stage(sv, dv, dsb)
            pltpu.async_copy(rows, agg_sh.at[dsb], ss_, add=True)
            @pl.when(kk + 2 < NCHUNK)
            def _():
                start_idx(kk + 2, sv, dv, si_)
            wait_scatter(rows, dsb, ss_)
            @pl.when(kk + 2 < NCHUNK)
            def _():
                wait_idx(sv, dv, si_)
                start_gather(sv, rows, sr_)

        def body(i, _):
            ka = 2 * i
            half(ka, srcA, dstA, dstSA, rows_a, sia, sra, ssa)
            half(ka + 1, srcB, dstB, dstSB, rows_b, sib, srb, ssb)
            return 0

        lax.fori_loop(0, (NCHUNK - 1) // 2, body, 0)
        half(NCHUNK - 1, srcA, dstA, dstSA, rows_a, sia, sra, ssa)
        plsc.subcore_barrier()

        def merge(e, _):
            off = e * EC
            for j in range(EC // 16):
                srcA[pl.ds(j * 16, 16)] = off + j * 16 + lax.iota(jnp.int32, 16)
            pltpu.sync_copy(hd_loc.at[pl.ds(off, EC)], hd_sh.at[srcA], add=True)
            pltpu.sync_copy(hs_loc.at[pl.ds(off, EC)], hs_sh.at[srcA], add=True)
            return 0

        lax.fori_loop(0, N // EC, merge, 0)
        plsc.subcore_barrier()

        pltpu.sync_copy(agg_sh.at[pl.ds(r0, ROWS_PER_TILE), :],
                        agg_out.at[c, pl.ds(r0, ROWS_PER_TILE), :])
        @pl.when(sid == 0)
        def _():
            pltpu.sync_copy(hd_sh, hist_out.at[c, 0, :])
            pltpu.sync_copy(hs_sh, hist_out.at[c, 1, :])

    return k(x, src, dst)


BN = 1000


def _dense_body(x_ref, a0_ref, a1_ref, cd0_ref, cd1_ref, cs0_ref, cs1_ref,
                si_ref,
                wl_ref, bl_ref, wr_ref, wr1_ref, br1_ref, wr2_ref, br2_ref,
                wdel_ref, wc_ref, wres_ref, wd_ref, bd_ref,
                scores_ref, delta_ref, res_ref, ucs_ref):
    i = pl.program_id(0)
    x = x_ref[...]
    cnt = cd0_ref[...] + cd1_ref[...]
    deg = cnt + cs0_ref[...] + cs1_ref[...]
    mean = (a0_ref[0] + a1_ref[0]) / jnp.maximum(cnt, 1.0)

    def mm(a, w_ref):
        return lax.dot_general(a, w_ref[...], (((1,), (1,)), ((), ())),
                               preferred_element_type=jnp.float32)

    pre = mm(mean, wl_ref) + bl_ref[...] + mm(x, wr_ref) + x
    x_gnn = 0.5 * pre * (1.0 + lax.erf(pre * 0.7071067811865476))
    h1 = jnp.maximum(mm(x_gnn, wr1_ref) + br1_ref[...], 0.0)
    sc = jnp.sum(h1 * wr2_ref[...], axis=1, keepdims=True) + br2_ref[0, 0]
    scores_ref[...] = sc + deg

    dpre = mm(x_gnn, wdel_ref)
    cc = mm(x_gnn, wc_ref)
    res_ref[...] = mm(x_gnn, wres_ref)
    z = mm(dpre, wd_ref) + bd_ref[...]
    delta_ref[...] = jnp.maximum(z, 0.0) + jnp.log1p(jnp.exp(-jnp.abs(z)))

    @pl.when(i == 0)
    def _():
        ucs_ref[...] = jnp.zeros_like(ucs_ref)
    ucs_ref[0:1, :] += jnp.sum(x_gnn * cc, axis=0, keepdims=True)
    ucs_ref[1:2, :] += jnp.sum(cc, axis=0, keepdims=True)


def _dense(x, aggp, hist, sinit, Wl, bl, Wr, Wr1, br1, Wr2, br2,
           Wdel, Wc, Wres, Wd, bd):
    grid = N // BN
    row = lambda i: (i, 0)
    full = lambda i: (0, 0)
    rspec = pl.BlockSpec((BN, D), row)
    aspec0 = pl.BlockSpec((1, BN, D), lambda i: (0, i, 0))
    aspec1 = pl.BlockSpec((1, BN, D), lambda i: (1, i, 0))
    cspec = pl.BlockSpec((BN, 1), row)
    return pl.pallas_call(
        _dense_body,
        grid=(grid,),
        in_specs=[rspec, aspec0, aspec1,
                  cspec, cspec, cspec, cspec,
                  pl.BlockSpec((BN, 1), row),
                  pl.BlockSpec((D, D), full), pl.BlockSpec((1, D), full),
                  pl.BlockSpec((D, D), full),
                  pl.BlockSpec((32, D), full), pl.BlockSpec((1, 32), full),
                  pl.BlockSpec((1, 32), full), pl.BlockSpec((1, 1), full),
                  pl.BlockSpec((D, D), full), pl.BlockSpec((D, D), full),
                  pl.BlockSpec((D, D), full), pl.BlockSpec((D, D), full),
                  pl.BlockSpec((1, D), full)],
        out_specs=[pl.BlockSpec((BN, 1), row), rspec, rspec,
                   pl.BlockSpec((8, D), full)],
        out_shape=[jax.ShapeDtypeStruct((NP, 1), jnp.float32),
                   jax.ShapeDtypeStruct((NP, D), jnp.float32),
                   jax.ShapeDtypeStruct((NP, D), jnp.float32),
                   jax.ShapeDtypeStruct((8, D), jnp.float32)],
        input_output_aliases={7: 0},
        compiler_params=pltpu.CompilerParams(
            dimension_semantics=("arbitrary",)),
    )(x, aggp, aggp, hist[0, 0, :, None], hist[1, 0, :, None],
      hist[0, 1, :, None], hist[1, 1, :, None], sinit, Wl, bl, Wr, Wr1, br1,
      Wr2, br2, Wdel, Wc, Wres, Wd, bd)
RB = 128
RC = 128


def _rank_body(si_ref, srow_ref, rank_ref):
    ib = pl.program_id(0)
    si = jnp.broadcast_to(si_ref[...], (RB, RC))

    one = jnp.float32(1.0)
    zero = jnp.float32(0.0)

    def pre1(k, acc):
        sj = jnp.broadcast_to(srow_ref[0:1, pl.ds(k * RC, RC)], (RB, RC))
        return acc + jnp.where(sj >= si, one, zero)

    def post1(k, acc):
        sj = jnp.broadcast_to(srow_ref[0:1, pl.ds(k * RC, RC)], (RB, RC))
        return acc + jnp.where(sj > si, one, zero)

    def pre2(t, acc):
        return pre1(2 * t + 1, pre1(2 * t, acc))

    acc = jnp.zeros((RB, RC), jnp.float32)
    acc = lax.fori_loop(0, ib // 2, pre2, acc)
    acc = lax.cond(ib % 2 == 1, lambda a: pre1(ib - 1, a), lambda a: a, acc)
    npost = (NP // RC) - ib - 1
    acc = lax.cond(npost % 2 == 1, lambda a: post1(ib + 1, a), lambda a: a, acc)
    p0 = ib + 1 + (npost % 2)

    def post2(t, acc):
        return post1(p0 + 2 * t + 1, post1(p0 + 2 * t, acc))

    acc = lax.fori_loop(0, npost // 2, post2, acc)
    sj = jnp.broadcast_to(srow_ref[0:1, pl.ds(ib * RC, RC)], (RB, RC))
    gi = lax.broadcasted_iota(jnp.int32, (RB, RC), 0)
    gj = lax.broadcasted_iota(jnp.int32, (RB, RC), 1)
    cmp = (sj > si) | ((sj == si) & (gj < gi))
    acc = acc + jnp.where(cmp, one, zero)
    rank_ref[...] = jnp.sum(acc, axis=1, keepdims=True).astype(jnp.int32)


def _rank(s_col, s_row):
    return pl.pallas_call(
        _rank_body,
        grid=(NP // RB,),
        in_specs=[pl.BlockSpec((RB, 1), lambda i: (i, 0)),
                  pl.BlockSpec((1, NP), lambda i: (0, 0))],
        out_specs=pl.BlockSpec((RB, 1), lambda i: (i, 0)),
        out_shape=jax.ShapeDtypeStruct((NP, 1), jnp.int32),
        compiler_params=pltpu.CompilerParams(
            dimension_semantics=("arbitrary",)),
    )(s_col, s_row)


RPW = NP // NW
RCH = 80


def _permute_rows(rows, rank, scatter: bool):
    mesh = plsc.VectorSubcoreMesh(core_axis_name="c", subcore_axis_name="s")

    @functools.partial(
        pl.kernel,
        out_type=jax.ShapeDtypeStruct((NP, D), jnp.float32),
        mesh=mesh,
        scratch_types=[
            pltpu.VMEM((RCH,), jnp.int32),
            pltpu.VMEM((RCH, D), jnp.float32),
            pltpu.SemaphoreType.DMA,
        ],
        compiler_params=pltpu.CompilerParams(needs_layout_passes=False),
    )
    def k(rows_hbm, rank_hbm, out_hbm, idx_v, buf_v, sem):
        c = lax.axis_index("c")
        sid = lax.axis_index("s")
        base = (sid * 2 + c) * RPW

        def body(e, _):
            off = base + e * RCH
            pltpu.sync_copy(rank_hbm.at[pl.ds(off, RCH)], idx_v)
            if scatter:
                pltpu.sync_copy(rows_hbm.at[pl.ds(off, RCH), :], buf_v)
                pltpu.async_copy(buf_v, out_hbm.at[idx_v], sem).wait()
            else:
                pltpu.async_copy(rows_hbm.at[idx_v], buf_v, sem).wait()
                pltpu.sync_copy(buf_v, out_hbm.at[pl.ds(off, RCH), :])
            return 0

        lax.fori_loop(0, RPW // RCH, body, 0)

    return k(rows, rank)


CB = 512


def _cumsum_body(x_ref, o_ref, carry_ref):
    i = pl.program_id(0)

    @pl.when(i == 0)
    def _():
        carry_ref[...] = jnp.zeros_like(carry_ref)

    blk = x_ref[...]
    ri = lax.broadcasted_iota(jnp.int32, (CB, CB), 0)
    ci = lax.broadcasted_iota(jnp.int32, (CB, CB), 1)
    L = (ri >= ci).astype(jnp.float32)
    c = carry_ref[0:1, :]
    o_ref[...] = lax.dot_general(L, blk, (((1,), (0,)), ((), ())),
                                 preferred_element_type=jnp.float32) + c
    carry_ref[0:1, :] = c + jnp.sum(blk, axis=0, keepdims=True)


def _cumsum(xs):
    return pl.pallas_call(
        _cumsum_body,
        grid=(NP // CB,),
        in_specs=[pl.BlockSpec((CB, D), lambda i: (i, 0))],
        out_specs=pl.BlockSpec((CB, D), lambda i: (i, 0)),
        out_shape=jax.ShapeDtypeStruct((NP, D), jnp.float32),
        scratch_shapes=[pltpu.VMEM((8, D), jnp.float32)],
        compiler_params=pltpu.CompilerParams(
            dimension_semantics=("arbitrary",)),
    )(xs)


def _final_body(x_ref, d_ref, p_ref, r_ref, ucs_ref, a_ref, bp_ref, dp_ref,
                o_ref):
    a = a_ref[...]
    bp = bp_ref[...]
    c1 = bp * ucs_ref[0:1, :]
    c2 = bp * ucs_ref[1:2, :]
    dp = dp_ref[0, 0]
    y = jnp.exp(d_ref[...] * a) * c1 + p_ref[...] * c2
    o = y + r_ref[...] * dp
    mu = jnp.mean(o, axis=1, keepdims=True)
    dev = o - mu
    var = jnp.mean(dev * dev, axis=1, keepdims=True)
    o_ref[...] = x_ref[...] + dev * lax.rsqrt(var + 1e-5)


def _final(x, delta, P, res, ucs, a, bp, dp):
    row = lambda i: (i, 0)
    full = lambda i: (0, 0)
    rspec = pl.BlockSpec((BN, D), row)
    return pl.pallas_call(
        _final_body,
        grid=(N // BN,),
        in_specs=[rspec, rspec, rspec, rspec,
                  pl.BlockSpec((8, D), full), pl.BlockSpec((1, D), full),
                  pl.BlockSpec((1, D), full), pl.BlockSpec((1, 1), full)],
        out_specs=rspec,
        out_shape=jax.ShapeDtypeStruct((N, D), jnp.float32),
        compiler_params=pltpu.CompilerParams(
            dimension_semantics=("arbitrary",)),
    )(x, delta, P, res, ucs, a, bp, dp)


def kernel(x, edge_index, Wl, bl, Wr, Wr1, br1, Wr2, br2, Wproj, A, Bp, Dp,
           Wd, bd):
    src = edge_index[0]
    dst = edge_index[1]

    aggp, hist = _edge_agg(x, src, dst)

    sinit = jnp.full((NP, 1), -jnp.inf, jnp.float32)
    scores, delta, res, ucs = _dense(
        x, aggp, hist, sinit,
        Wl, bl.reshape(1, D), Wr, Wr1, br1.reshape(1, 32),
        Wr2, br2.reshape(1, 1),
        Wproj[:D], Wproj[2 * D:3 * D], Wproj[3 * D:], Wd, bd.reshape(1, D))

    rank = _rank(scores, scores.reshape(1, NP))
    rank_flat = rank.reshape(NP)

    ds = _permute_rows(delta, rank_flat, scatter=True)
    Ps = _cumsum(ds)
    Ppad = _permute_rows(Ps, rank_flat, scatter=False)

    return _final(x, delta, Ppad, res, ucs,
                  A.reshape(1, D), Bp.reshape(1, D), Dp.reshape(1, 1))


# speedup vs baseline: 1.2262x; 1.0310x over previous
import functools

import jax
import jax.numpy as jnp
from jax import lax
from jax.experimental import pallas as pl
from jax.experimental.pallas import tpu as pltpu
from jax.experimental.pallas import tpu_sc as plsc

N = 10000
E = 320000
D = 128
NP = 10240
NW = 32
EPW = E // NW
EC = 80
NCHUNK = EPW // EC
ROWS_PER_TILE = NP // 16
ZB = 128


def _edge_agg(x, edge_index):
    mesh = plsc.VectorSubcoreMesh(core_axis_name="c", subcore_axis_name="s")

    @functools.partial(
        pl.kernel,
        out_type=[
            jax.ShapeDtypeStruct((2, NP, D), jnp.float32),
            jax.ShapeDtypeStruct((2, 2, N), jnp.float32),
        ],
        mesh=mesh,
        scratch_types=[
            pltpu.VMEM((EC,), jnp.int32),
            pltpu.VMEM((EC,), jnp.int32),
            pltpu.VMEM((EC,), jnp.int32),
            pltpu.VMEM((EC,), jnp.int32),
            pltpu.VMEM((EC,), jnp.int32),
            pltpu.VMEM((EC,), jnp.int32),
            pltpu.VMEM((EC, D), jnp.float32),
            pltpu.VMEM((EC, D), jnp.float32),
            pltpu.VMEM((N,), jnp.float32),
            pltpu.VMEM((N,), jnp.float32),
            pltpu.VMEM_SHARED((NP, D), jnp.float32),
            pltpu.VMEM_SHARED((N,), jnp.float32),
            pltpu.VMEM_SHARED((N,), jnp.float32),
            pltpu.SemaphoreType.DMA,
            pltpu.SemaphoreType.DMA,
            pltpu.SemaphoreType.DMA,
            pltpu.SemaphoreType.DMA,
            pltpu.SemaphoreType.DMA,
            pltpu.SemaphoreType.DMA,
        ],
        compiler_params=pltpu.CompilerParams(needs_layout_passes=False),
    )
    def k(x_hbm, ei_hbm, agg_out, hist_out,
          srcA, dstA, srcB, dstB, dstSA, dstSB, rows_a, rows_b, hd_loc, hs_loc,
          agg_sh, hd_sh, hs_sh, sia, sib, sra, srb, ssa, ssb):
        c = lax.axis_index("c")
        sid = lax.axis_index("s")
        z16 = jnp.zeros((16,), jnp.float32)
        ones16 = jnp.ones((16,), jnp.float32)
        wid = sid * 2 + c

        def start_idx(kk, sv, dv, sem):
            off = pl.multiple_of(wid * EPW + kk * EC, 8)
            pltpu.async_copy(ei_hbm.at[pl.ds(off, EC)], sv, sem)
            pltpu.async_copy(ei_hbm.at[pl.ds(E + off, EC)], dv, sem)

        def wait_idx(sv, dv, sem):
            pltpu.make_async_copy(ei_hbm.at[pl.ds(0, EC)], sv, sem).wait()
            pltpu.make_async_copy(ei_hbm.at[pl.ds(0, EC)], dv, sem).wait()

        def start_gather(sv, buf, sem):
            pltpu.async_copy(x_hbm.at[sv], buf, sem)

        def wait_gather(buf, sem):
            pltpu.make_async_copy(x_hbm.at[srcA], buf, sem).wait()

        def hists_and_stage(sv, dv, ds_buf):
            for j in range(EC // 16):
                di = dv[pl.ds(j * 16, 16)]
                si = sv[pl.ds(j * 16, 16)]
                ds_buf[pl.ds(j * 16, 16)] = di
                plsc.addupdate_scatter(hd_loc, [di], ones16)
                plsc.addupdate_scatter(hs_loc, [si], ones16)

        def wait_scatter(rows, ds_buf, sem):
            pltpu.make_async_copy(rows, agg_sh.at[ds_buf], sem).wait()

        start_idx(0, srcA, dstA, sia)
        start_idx(1, srcB, dstB, sib)

        def zloop(i, _):
            hd_loc[pl.ds(i * 16, 16)] = z16
            hs_loc[pl.ds(i * 16, 16)] = z16
            return 0
        lax.fori_loop(0, N // 16, zloop, 0)

        def zloop2(i, _):
            for j in range(D // 16):
                rows_a[i, pl.ds(j * 16, 16)] = z16
            return 0
        lax.fori_loop(0, EC, zloop2, 0)

        r0 = sid * ROWS_PER_TILE
        for t in range(ROWS_PER_TILE // EC):
            pltpu.sync_copy(rows_a, agg_sh.at[pl.ds(r0 + t * EC, EC), :])
        @pl.when(sid == 0)
        def _():
            pltpu.sync_copy(hd_loc, hd_sh)
            pltpu.sync_copy(hs_loc, hs_sh)

        plsc.subcore_barrier()

        wait_idx(srcA, dstA, sia)
        start_gather(srcA, rows_a, sra)
        wait_idx(srcB, dstB, sib)
        start_gather(srcB, rows_b, srb)

        def half(kk, sv, dv, dsb, rows, si_, sr_, ss_):
            wait_gather(rows, sr_)
            hists_and_stage(sv, dv, dsb)
            pltpu.async_copy(rows, agg_sh.at[dsb], ss_, add=True)
            @pl.when(kk + 2 < NCHUNK)
            def _():
                start_idx(kk + 2, sv, dv, si_)
            wait_scatter(rows, dsb, ss_)
            @pl.when(kk + 2 < NCHUNK)
            def _():
                wait_idx(sv, dv, si_)
                start_gather(sv, rows, sr_)

        def body(i, _):
            ka = 2 * i
            half(ka, srcA, dstA, dstSA, rows_a, sia, sra, ssa)
            half(ka + 1, srcB, dstB, dstSB, rows_b, sib, srb, ssb)
            return 0

        lax.fori_loop(0, (NCHUNK - 1) // 2, body, 0)
        half(NCHUNK - 1, srcA, dstA, dstSA, rows_a, sia, sra, ssa)
        plsc.subcore_barrier()

        def merge(e, _):
            off = e * EC
            for j in range(EC // 16):
                srcA[pl.ds(j * 16, 16)] = off + j * 16 + lax.iota(jnp.int32, 16)
            pltpu.sync_copy(hd_loc.at[pl.ds(off, EC)], hd_sh.at[srcA], add=True)
            pltpu.sync_copy(hs_loc.at[pl.ds(off, EC)], hs_sh.at[srcA], add=True)
            return 0

        lax.fori_loop(0, N // EC, merge, 0)
        plsc.subcore_barrier()

        pltpu.sync_copy(agg_sh.at[pl.ds(r0, ROWS_PER_TILE), :],
                        agg_out.at[c, pl.ds(r0, ROWS_PER_TILE), :])
        @pl.when(sid == 0)
        def _():
            pltpu.sync_copy(hd_sh, hist_out.at[c, 0, :])
            pltpu.sync_copy(hs_sh, hist_out.at[c, 1, :])

    return k(x, edge_index.reshape(2 * E))


BN = 1000


def _dense_body(x_ref, a0_ref, a1_ref, cd0_ref, cd1_ref, cs0_ref, cs1_ref,
                si_ref,
                wl_ref, bl_ref, wr_ref, wr1_ref, br1_ref, wr2_ref, br2_ref,
                wdel_ref, wc_ref, wres_ref, wd_ref, bd_ref,
                scores_ref, delta_ref, res_ref, ucs_ref):
    i = pl.program_id(0)
    x = x_ref[...]
    cnt = cd0_ref[...] + cd1_ref[...]
    deg = cnt + cs0_ref[...] + cs1_ref[...]
    mean = (a0_ref[0] + a1_ref[0]) / jnp.maximum(cnt, 1.0)

    def mm(a, w_ref):
        return lax.dot_general(a, w_ref[...], (((1,), (1,)), ((), ())),
                               preferred_element_type=jnp.float32)

    pre = mm(mean, wl_ref) + bl_ref[...] + mm(x, wr_ref) + x
    x_gnn = 0.5 * pre * (1.0 + lax.erf(pre * 0.7071067811865476))
    h1 = jnp.maximum(mm(x_gnn, wr1_ref) + br1_ref[...], 0.0)
    sc = jnp.sum(h1 * wr2_ref[...], axis=1, keepdims=True) + br2_ref[0, 0]
    scores_ref[...] = sc + deg

    dpre = mm(x_gnn, wdel_ref)
    cc = mm(x_gnn, wc_ref)
    res_ref[...] = mm(x_gnn, wres_ref)
    z = mm(dpre, wd_ref) + bd_ref[...]
    delta_ref[...] = jnp.maximum(z, 0.0) + jnp.log1p(jnp.exp(-jnp.abs(z)))

    @pl.when(i == 0)
    def _():
        ucs_ref[...] = jnp.zeros_like(ucs_ref)
    ucs_ref[0:1, :] += jnp.sum(x_gnn * cc, axis=0, keepdims=True)
    ucs_ref[1:2, :] += jnp.sum(cc, axis=0, keepdims=True)


def _dense(x, aggp, hist, sinit, Wl, bl, Wr, Wr1, br1, Wr2, br2,
           Wdel, Wc, Wres, Wd, bd):
    grid = N // BN
    row = lambda i: (i, 0)
    full = lambda i: (0, 0)
    rspec = pl.BlockSpec((BN, D), row)
    aspec0 = pl.BlockSpec((1, BN, D), lambda i: (0, i, 0))
    aspec1 = pl.BlockSpec((1, BN, D), lambda i: (1, i, 0))
    cspec = pl.BlockSpec((BN, 1), row)
    return pl.pallas_call(
        _dense_body,
        grid=(grid,),
        in_specs=[rspec, aspec0, aspec1,
                  cspec, cspec, cspec, cspec,
                  pl.BlockSpec((BN, 1), row),
                  pl.BlockSpec((D, D), full), pl.BlockSpec((1, D), full),
                  pl.BlockSpec((D, D), full),
                  pl.BlockSpec((32, D), full), pl.BlockSpec((1, 32), full),
                  pl.BlockSpec((1, 32), full), pl.BlockSpec((1, 1), full),
                  pl.BlockSpec((D, D), full), pl.BlockSpec((D, D), full),
                  pl.BlockSpec((D, D), full), pl.BlockSpec((D, D), full),
                  pl.BlockSpec((1, D), full)],
        out_specs=[pl.BlockSpec((BN, 1), row), rspec, rspec,
                   pl.BlockSpec((8, D), full)],
        out_shape=[jax.ShapeDtypeStruct((NP, 1), jnp.float32),
                   jax.ShapeDtypeStruct((NP, D), jnp.float32),
                   jax.ShapeDtypeStruct((NP, D), jnp.float32),
                   jax.ShapeDtypeStruct((8, D), jnp.float32)],
        input_output_aliases={7: 0},
        compiler_params=pltpu.CompilerParams(
            dimension_semantics=("arbitrary",)),
    )(x, aggp, aggp, hist[0, 0, :, None], hist[1, 0, :, None],
      hist[0, 1, :, None], hist[1, 1, :, None], sinit, Wl, bl, Wr, Wr1, br1,
      Wr2, br2, Wdel, Wc, Wres, Wd, bd)
RB = 128
RC = 128


def _rank_body(si_ref, srow_ref, rank_ref):
    ib = pl.program_id(0)
    si = jnp.broadcast_to(si_ref[...], (RB, RC))

    one = jnp.float32(1.0)
    zero = jnp.float32(0.0)

    def pre1(k, acc):
        sj = jnp.broadcast_to(srow_ref[0:1, pl.ds(k * RC, RC)], (RB, RC))
        return acc + jnp.where(sj >= si, one, zero)

    def post1(k, acc):
        sj = jnp.broadcast_to(srow_ref[0:1, pl.ds(k * RC, RC)], (RB, RC))
        return acc + jnp.where(sj > si, one, zero)

    def pre2(t, acc):
        return pre1(2 * t + 1, pre1(2 * t, acc))

    acc = jnp.zeros((RB, RC), jnp.float32)
    acc = lax.fori_loop(0, ib // 2, pre2, acc)
    acc = lax.cond(ib % 2 == 1, lambda a: pre1(ib - 1, a), lambda a: a, acc)
    npost = (NP // RC) - ib - 1
    acc = lax.cond(npost % 2 == 1, lambda a: post1(ib + 1, a), lambda a: a, acc)
    p0 = ib + 1 + (npost % 2)

    def post2(t, acc):
        return post1(p0 + 2 * t + 1, post1(p0 + 2 * t, acc))

    acc = lax.fori_loop(0, npost // 2, post2, acc)
    sj = jnp.broadcast_to(srow_ref[0:1, pl.ds(ib * RC, RC)], (RB, RC))
    gi = lax.broadcasted_iota(jnp.int32, (RB, RC), 0)
    gj = lax.broadcasted_iota(jnp.int32, (RB, RC), 1)
    cmp = (sj > si) | ((sj == si) & (gj < gi))
    acc = acc + jnp.where(cmp, one, zero)
    rank_ref[...] = jnp.sum(acc, axis=1, keepdims=True).astype(jnp.int32)


def _rank(s_col, s_row):
    return pl.pallas_call(
        _rank_body,
        grid=(NP // RB,),
        in_specs=[pl.BlockSpec((RB, 1), lambda i: (i, 0)),
                  pl.BlockSpec((1, NP), lambda i: (0, 0))],
        out_specs=pl.BlockSpec((RB, 1), lambda i: (i, 0)),
        out_shape=jax.ShapeDtypeStruct((NP, 1), jnp.int32),
        compiler_params=pltpu.CompilerParams(
            dimension_semantics=("arbitrary",)),
    )(s_col, s_row)


RPW = NP // NW
RCH = 80


def _permute_rows(rows, rank, scatter: bool):
    mesh = plsc.VectorSubcoreMesh(core_axis_name="c", subcore_axis_name="s")

    @functools.partial(
        pl.kernel,
        out_type=jax.ShapeDtypeStruct((NP, D), jnp.float32),
        mesh=mesh,
        scratch_types=[
            pltpu.VMEM((RCH,), jnp.int32),
            pltpu.VMEM((RCH, D), jnp.float32),
            pltpu.SemaphoreType.DMA,
        ],
        compiler_params=pltpu.CompilerParams(needs_layout_passes=False),
    )
    def k(rows_hbm, rank_hbm, out_hbm, idx_v, buf_v, sem):
        c = lax.axis_index("c")
        sid = lax.axis_index("s")
        base = (sid * 2 + c) * RPW

        def body(e, _):
            off = base + e * RCH
            pltpu.sync_copy(rank_hbm.at[pl.ds(off, RCH)], idx_v)
            if scatter:
                pltpu.sync_copy(rows_hbm.at[pl.ds(off, RCH), :], buf_v)
                pltpu.async_copy(buf_v, out_hbm.at[idx_v], sem).wait()
            else:
                pltpu.async_copy(rows_hbm.at[idx_v], buf_v, sem).wait()
                pltpu.sync_copy(buf_v, out_hbm.at[pl.ds(off, RCH), :])
            return 0

        lax.fori_loop(0, RPW // RCH, body, 0)

    return k(rows, rank)


CB = 512


def _cumsum_body(x_ref, o_ref, carry_ref):
    i = pl.program_id(0)

    @pl.when(i == 0)
    def _():
        carry_ref[...] = jnp.zeros_like(carry_ref)

    blk = x_ref[...]
    ri = lax.broadcasted_iota(jnp.int32, (CB, CB), 0)
    ci = lax.broadcasted_iota(jnp.int32, (CB, CB), 1)
    L = (ri >= ci).astype(jnp.float32)
    c = carry_ref[0:1, :]
    o_ref[...] = lax.dot_general(L, blk, (((1,), (0,)), ((), ())),
                                 preferred_element_type=jnp.float32) + c
    carry_ref[0:1, :] = c + jnp.sum(blk, axis=0, keepdims=True)


def _cumsum(xs):
    return pl.pallas_call(
        _cumsum_body,
        grid=(NP // CB,),
        in_specs=[pl.BlockSpec((CB, D), lambda i: (i, 0))],
        out_specs=pl.BlockSpec((CB, D), lambda i: (i, 0)),
        out_shape=jax.ShapeDtypeStruct((NP, D), jnp.float32),
        scratch_shapes=[pltpu.VMEM((8, D), jnp.float32)],
        compiler_params=pltpu.CompilerParams(
            dimension_semantics=("arbitrary",)),
    )(xs)


def _final_body(x_ref, d_ref, p_ref, r_ref, ucs_ref, a_ref, bp_ref, dp_ref,
                o_ref):
    a = a_ref[...]
    bp = bp_ref[...]
    c1 = bp * ucs_ref[0:1, :]
    c2 = bp * ucs_ref[1:2, :]
    dp = dp_ref[0, 0]
    y = jnp.exp(d_ref[...] * a) * c1 + p_ref[...] * c2
    o = y + r_ref[...] * dp
    mu = jnp.mean(o, axis=1, keepdims=True)
    dev = o - mu
    var = jnp.mean(dev * dev, axis=1, keepdims=True)
    o_ref[...] = x_ref[...] + dev * lax.rsqrt(var + 1e-5)


def _final(x, delta, P, res, ucs, a, bp, dp):
    row = lambda i: (i, 0)
    full = lambda i: (0, 0)
    rspec = pl.BlockSpec((BN, D), row)
    return pl.pallas_call(
        _final_body,
        grid=(N // BN,),
        in_specs=[rspec, rspec, rspec, rspec,
                  pl.BlockSpec((8, D), full), pl.BlockSpec((1, D), full),
                  pl.BlockSpec((1, D), full), pl.BlockSpec((1, 1), full)],
        out_specs=rspec,
        out_shape=jax.ShapeDtypeStruct((N, D), jnp.float32),
        compiler_params=pltpu.CompilerParams(
            dimension_semantics=("arbitrary",)),
    )(x, delta, P, res, ucs, a, bp, dp)


def kernel(x, edge_index, Wl, bl, Wr, Wr1, br1, Wr2, br2, Wproj, A, Bp, Dp,
           Wd, bd):
    aggp, hist = _edge_agg(x, edge_index)

    sinit = jnp.full((NP, 1), -jnp.inf, jnp.float32)
    scores, delta, res, ucs = _dense(
        x, aggp, hist, sinit,
        Wl, bl.reshape(1, D), Wr, Wr1, br1.reshape(1, 32),
        Wr2, br2.reshape(1, 1),
        Wproj[:D], Wproj[2 * D:3 * D], Wproj[3 * D:], Wd, bd.reshape(1, D))

    rank = _rank(scores, scores.reshape(1, NP))
    rank_flat = rank.reshape(NP)

    ds = _permute_rows(delta, rank_flat, scatter=True)
    Ps = _cumsum(ds)
    Ppad = _permute_rows(Ps, rank_flat, scatter=False)

    return _final(x, delta, Ppad, res, ucs,
                  A.reshape(1, D), Bp.reshape(1, D), Dp.reshape(1, 1))


# speedup vs baseline: 1.2756x; 1.0403x over previous
import functools

import jax
import jax.numpy as jnp
from jax import lax
from jax.experimental import pallas as pl
from jax.experimental.pallas import tpu as pltpu
from jax.experimental.pallas import tpu_sc as plsc

N = 10000
E = 320000
D = 128
NP = 10240
NW = 32
EPW = E // NW
EC = 112
NFULL = EPW // EC
ECT = EPW - NFULL * EC
MC = 80
ROWS_PER_TILE = 640
ZR = 80


def _edge_agg(x, edge_index):
    mesh = plsc.VectorSubcoreMesh(core_axis_name="c", subcore_axis_name="s")

    @functools.partial(
        pl.kernel,
        out_type=[
            jax.ShapeDtypeStruct((2, N, D), jnp.float32),
            jax.ShapeDtypeStruct((2, 2, N), jnp.float32),
        ],
        mesh=mesh,
        scratch_types=[
            pltpu.VMEM((EC,), jnp.int32),
            pltpu.VMEM((EC,), jnp.int32),
            pltpu.VMEM((EC,), jnp.int32),
            pltpu.VMEM((EC,), jnp.int32),
            pltpu.VMEM((EC,), jnp.int32),
            pltpu.VMEM((ECT,), jnp.int32),
            pltpu.VMEM((MC,), jnp.int32),
            pltpu.VMEM((EC, D), jnp.float32),
            pltpu.VMEM((EC, D), jnp.float32),
            pltpu.VMEM((N,), jnp.float32),
            pltpu.VMEM((N,), jnp.float32),
            pltpu.VMEM_SHARED((N, D), jnp.float32),
            pltpu.VMEM_SHARED((N,), jnp.float32),
            pltpu.VMEM_SHARED((N,), jnp.float32),
            pltpu.SemaphoreType.DMA,
            pltpu.SemaphoreType.DMA,
            pltpu.SemaphoreType.DMA,
            pltpu.SemaphoreType.DMA,
            pltpu.SemaphoreType.DMA,
            pltpu.SemaphoreType.DMA,
        ],
        compiler_params=pltpu.CompilerParams(needs_layout_passes=False),
    )
    def k(x_hbm, ei_hbm, agg_out, hist_out,
          srcA, dstA, srcB, dstB, dstS, dstT, mrgI, rows_a, rows_b,
          hd_loc, hs_loc,
          agg_sh, hd_sh, hs_sh, sia, sib, sra, srb, ssa, ssb):
        c = lax.axis_index("c")
        sid = lax.axis_index("s")
        z16 = jnp.zeros((16,), jnp.float32)
        ones16 = jnp.ones((16,), jnp.float32)
        wid = sid * 2 + c

        def start_idx(kk, sv, dv, sem):
            off = pl.multiple_of(wid * EPW + kk * EC, 8)
            pltpu.async_copy(ei_hbm.at[pl.ds(off, EC)], sv, sem)
            pltpu.async_copy(ei_hbm.at[pl.ds(E + off, EC)], dv, sem)

        def wait_idx(sv, dv, sem):
            pltpu.make_async_copy(ei_hbm.at[pl.ds(0, EC)], sv, sem).wait()
            pltpu.make_async_copy(ei_hbm.at[pl.ds(0, EC)], dv, sem).wait()

        def start_gather(sv, buf, sem):
            pltpu.async_copy(x_hbm.at[sv], buf, sem)

        def wait_gather(buf, sem):
            pltpu.make_async_copy(x_hbm.at[srcA], buf, sem).wait()

        def hists_and_stage(sv, dv, ds_buf):
            for j in range(EC // 16):
                di = dv[pl.ds(j * 16, 16)]
                si = sv[pl.ds(j * 16, 16)]
                ds_buf[pl.ds(j * 16, 16)] = di
                plsc.addupdate_scatter(hd_loc, [di], ones16)
                plsc.addupdate_scatter(hs_loc, [si], ones16)

        def wait_scatter(rows, ds_buf, sem):
            pltpu.make_async_copy(rows, agg_sh.at[ds_buf], sem).wait()

        start_idx(0, srcA, dstA, sia)
        start_idx(1, srcB, dstB, sib)

        def zloop(i, _):
            hd_loc[pl.ds(i * 16, 16)] = z16
            hs_loc[pl.ds(i * 16, 16)] = z16
            return 0
        lax.fori_loop(0, N // 16, zloop, 0)

        def zloop2(i, _):
            for j in range(D // 16):
                rows_a[i, pl.ds(j * 16, 16)] = z16
            return 0
        lax.fori_loop(0, EC, zloop2, 0)

        r0 = sid * ROWS_PER_TILE
        @pl.when(sid < 15)
        def _():
            for t in range(ROWS_PER_TILE // ZR):
                pltpu.sync_copy(rows_a.at[0:ZR, :],
                                agg_sh.at[pl.ds(r0 + t * ZR, ZR), :])
        @pl.when(sid == 15)
        def _():
            for t in range((N - 15 * ROWS_PER_TILE) // ZR):
                pltpu.sync_copy(rows_a.at[0:ZR, :],
                                agg_sh.at[pl.ds(r0 + t * ZR, ZR), :])
        @pl.when(sid == 0)
        def _():
            pltpu.sync_copy(hd_loc, hd_sh)
            pltpu.sync_copy(hs_loc, hs_sh)

        plsc.subcore_barrier()

        wait_idx(srcA, dstA, sia)
        start_gather(srcA, rows_a, sra)
        wait_idx(srcB, dstB, sib)
        start_gather(srcB, rows_b, srb)

        def half(kk, sv, dv, dsb, rows, si_, sr_, ss_):
            wait_gather(rows, sr_)
            hists_and_stage(sv, dv, dsb)
            pltpu.async_copy(rows, agg_sh.at[dsb], ss_, add=True)
            @pl.when(kk + 2 < NFULL)
            def _():
                start_idx(kk + 2, sv, dv, si_)
            wait_scatter(rows, dsb, ss_)
            @pl.when(kk + 2 < NFULL)
            def _():
                wait_idx(sv, dv, si_)
                start_gather(sv, rows, sr_)

        def body(i, _):
            ka = 2 * i
            half(ka, srcA, dstA, dstS, rows_a, sia, sra, ssa)
            half(ka + 1, srcB, dstB, dstS, rows_b, sib, srb, ssb)
            return 0

        lax.fori_loop(0, NFULL // 2, body, 0)
        half(NFULL - 1, srcA, dstA, dstS, rows_a, sia, sra, ssa)
        off_t = pl.multiple_of(wid * EPW + NFULL * EC, 8)
        pltpu.async_copy(ei_hbm.at[pl.ds(off_t, ECT)], srcA.at[0:ECT], sia)
        pltpu.async_copy(ei_hbm.at[pl.ds(E + off_t, ECT)], dstA.at[0:ECT], sib)
        pltpu.make_async_copy(ei_hbm.at[pl.ds(0, ECT)], srcA.at[0:ECT], sia).wait()
        pltpu.make_async_copy(ei_hbm.at[pl.ds(0, ECT)], dstA.at[0:ECT], sib).wait()
        pltpu.async_copy(x_hbm.at[srcA.at[0:ECT]], rows_a.at[0:ECT, :], sra)
        pltpu.make_async_copy(x_hbm.at[srcA.at[0:ECT]], rows_a.at[0:ECT, :], sra).wait()
        for j in range(ECT // 16):
            di = dstA[pl.ds(j * 16, 16)]
            si = srcA[pl.ds(j * 16, 16)]
            dstT[pl.ds(j * 16, 16)] = di
            plsc.addupdate_scatter(hd_loc, [di], ones16)
            plsc.addupdate_scatter(hs_loc, [si], ones16)
        pltpu.sync_copy(rows_a.at[0:ECT, :], agg_sh.at[dstT], add=True)
        plsc.subcore_barrier()

        def merge(e, _):
            off = e * MC
            for j in range(MC // 16):
                mrgI[pl.ds(j * 16, 16)] = off + j * 16 + lax.iota(jnp.int32, 16)
            pltpu.sync_copy(hd_loc.at[pl.ds(off, MC)], hd_sh.at[mrgI], add=True)
            pltpu.sync_copy(hs_loc.at[pl.ds(off, MC)], hs_sh.at[mrgI], add=True)
            return 0

        lax.fori_loop(0, N // MC, merge, 0)
        plsc.subcore_barrier()

        @pl.when(sid < 15)
        def _():
            pltpu.sync_copy(agg_sh.at[pl.ds(r0, ROWS_PER_TILE), :],
                            agg_out.at[c, pl.ds(r0, ROWS_PER_TILE), :])
        @pl.when(sid == 15)
        def _():
            pltpu.sync_copy(agg_sh.at[pl.ds(r0, N - 15 * ROWS_PER_TILE), :],
                            agg_out.at[c, pl.ds(r0, N - 15 * ROWS_PER_TILE), :])
        @pl.when(sid == 0)
        def _():
            pltpu.sync_copy(hd_sh, hist_out.at[c, 0, :])
            pltpu.sync_copy(hs_sh, hist_out.at[c, 1, :])

    return k(x, edge_index.reshape(2 * E))


BN = 1000


def _dense_body(x_ref, a0_ref, a1_ref, cd0_ref, cd1_ref, cs0_ref, cs1_ref,
                si_ref,
                wl_ref, bl_ref, wr_ref, wr1_ref, br1_ref, wr2_ref, br2_ref,
                wdel_ref, wc_ref, wres_ref, wd_ref, bd_ref,
                scores_ref, delta_ref, res_ref, ucs_ref):
    i = pl.program_id(0)
    x = x_ref[...]
    cnt = cd0_ref[...] + cd1_ref[...]
    deg = cnt + cs0_ref[...] + cs1_ref[...]
    mean = (a0_ref[0] + a1_ref[0]) / jnp.maximum(cnt, 1.0)

    def mm(a, w_ref):
        return lax.dot_general(a, w_ref[...], (((1,), (1,)), ((), ())),
                               preferred_element_type=jnp.float32)

    pre = mm(mean, wl_ref) + bl_ref[...] + mm(x, wr_ref) + x
    x_gnn = 0.5 * pre * (1.0 + lax.erf(pre * 0.7071067811865476))
    h1 = jnp.maximum(mm(x_gnn, wr1_ref) + br1_ref[...], 0.0)
    sc = jnp.sum(h1 * wr2_ref[...], axis=1, keepdims=True) + br2_ref[0, 0]
    scores_ref[...] = sc + deg

    dpre = mm(x_gnn, wdel_ref)
    cc = mm(x_gnn, wc_ref)
    res_ref[...] = mm(x_gnn, wres_ref)
    z = mm(dpre, wd_ref) + bd_ref[...]
    delta_ref[...] = jnp.maximum(z, 0.0) + jnp.log1p(jnp.exp(-jnp.abs(z)))

    @pl.when(i == 0)
    def _():
        ucs_ref[...] = jnp.zeros_like(ucs_ref)
    ucs_ref[0:1, :] += jnp.sum(x_gnn * cc, axis=0, keepdims=True)
    ucs_ref[1:2, :] += jnp.sum(cc, axis=0, keepdims=True)


def _dense(x, aggp, hist, sinit, Wl, bl, Wr, Wr1, br1, Wr2, br2,
           Wdel, Wc, Wres, Wd, bd):
    grid = N // BN
    row = lambda i: (i, 0)
    full = lambda i: (0, 0)
    rspec = pl.BlockSpec((BN, D), row)
    aspec0 = pl.BlockSpec((1, BN, D), lambda i: (0, i, 0))
    aspec1 = pl.BlockSpec((1, BN, D), lambda i: (1, i, 0))
    cspec = pl.BlockSpec((BN, 1), row)
    return pl.pallas_call(
        _dense_body,
        grid=(grid,),
        in_specs=[rspec, aspec0, aspec1,
                  cspec, cspec, cspec, cspec,
                  pl.BlockSpec((BN, 1), row),
                  pl.BlockSpec((D, D), full), pl.BlockSpec((1, D), full),
                  pl.BlockSpec((D, D), full),
                  pl.BlockSpec((32, D), full), pl.BlockSpec((1, 32), full),
                  pl.BlockSpec((1, 32), full), pl.BlockSpec((1, 1), full),
                  pl.BlockSpec((D, D), full), pl.BlockSpec((D, D), full),
                  pl.BlockSpec((D, D), full), pl.BlockSpec((D, D), full),
                  pl.BlockSpec((1, D), full)],
        out_specs=[pl.BlockSpec((BN, 1), row), rspec, rspec,
                   pl.BlockSpec((8, D), full)],
        out_shape=[jax.ShapeDtypeStruct((NP, 1), jnp.float32),
                   jax.ShapeDtypeStruct((NP, D), jnp.float32),
                   jax.ShapeDtypeStruct((NP, D), jnp.float32),
                   jax.ShapeDtypeStruct((8, D), jnp.float32)],
        input_output_aliases={7: 0},
        compiler_params=pltpu.CompilerParams(
            dimension_semantics=("arbitrary",)),
    )(x, aggp, aggp, hist[0, 0, :, None], hist[1, 0, :, None],
      hist[0, 1, :, None], hist[1, 1, :, None], sinit, Wl, bl, Wr, Wr1, br1,
      Wr2, br2, Wdel, Wc, Wres, Wd, bd)
RB = 128
RC = 128


def _rank_body(si_ref, srow_ref, rank_ref):
    ib = pl.program_id(0)
    si = jnp.broadcast_to(si_ref[...], (RB, RC))

    one = jnp.float32(1.0)
    zero = jnp.float32(0.0)

    def pre1(k, acc):
        sj = jnp.broadcast_to(srow_ref[0:1, pl.ds(k * RC, RC)], (RB, RC))
        return acc + jnp.where(sj >= si, one, zero)

    def post1(k, acc):
        sj = jnp.broadcast_to(srow_ref[0:1, pl.ds(k * RC, RC)], (RB, RC))
        return acc + jnp.where(sj > si, one, zero)

    def pre2(t, acc):
        return pre1(2 * t + 1, pre1(2 * t, acc))

    acc = jnp.zeros((RB, RC), jnp.float32)
    acc = lax.fori_loop(0, ib // 2, pre2, acc)
    acc = lax.cond(ib % 2 == 1, lambda a: pre1(ib - 1, a), lambda a: a, acc)
    npost = (NP // RC) - ib - 1
    acc = lax.cond(npost % 2 == 1, lambda a: post1(ib + 1, a), lambda a: a, acc)
    p0 = ib + 1 + (npost % 2)

    def post2(t, acc):
        return post1(p0 + 2 * t + 1, post1(p0 + 2 * t, acc))

    acc = lax.fori_loop(0, npost // 2, post2, acc)
    sj = jnp.broadcast_to(srow_ref[0:1, pl.ds(ib * RC, RC)], (RB, RC))
    gi = lax.broadcasted_iota(jnp.int32, (RB, RC), 0)
    gj = lax.broadcasted_iota(jnp.int32, (RB, RC), 1)
    cmp = (sj > si) | ((sj == si) & (gj < gi))
    acc = acc + jnp.where(cmp, one, zero)
    rank_ref[...] = jnp.sum(acc, axis=1, keepdims=True).astype(jnp.int32)


def _rank(s_col, s_row):
    return pl.pallas_call(
        _rank_body,
        grid=(NP // RB,),
        in_specs=[pl.BlockSpec((RB, 1), lambda i: (i, 0)),
                  pl.BlockSpec((1, NP), lambda i: (0, 0))],
        out_specs=pl.BlockSpec((RB, 1), lambda i: (i, 0)),
        out_shape=jax.ShapeDtypeStruct((NP, 1), jnp.int32),
        compiler_params=pltpu.CompilerParams(
            dimension_semantics=("arbitrary",)),
    )(s_col, s_row)


RPW = NP // NW
RCH = 80


def _permute_rows(rows, rank, scatter: bool):
    mesh = plsc.VectorSubcoreMesh(core_axis_name="c", subcore_axis_name="s")

    @functools.partial(
        pl.kernel,
        out_type=jax.ShapeDtypeStruct((NP, D), jnp.float32),
        mesh=mesh,
        scratch_types=[
            pltpu.VMEM((RCH,), jnp.int32),
            pltpu.VMEM((RCH, D), jnp.float32),
            pltpu.SemaphoreType.DMA,
        ],
        compiler_params=pltpu.CompilerParams(needs_layout_passes=False),
    )
    def k(rows_hbm, rank_hbm, out_hbm, idx_v, buf_v, sem):
        c = lax.axis_index("c")
        sid = lax.axis_index("s")
        base = (sid * 2 + c) * RPW

        def body(e, _):
            off = base + e * RCH
            pltpu.sync_copy(rank_hbm.at[pl.ds(off, RCH)], idx_v)
            if scatter:
                pltpu.sync_copy(rows_hbm.at[pl.ds(off, RCH), :], buf_v)
                pltpu.async_copy(buf_v, out_hbm.at[idx_v], sem).wait()
            else:
                pltpu.async_copy(rows_hbm.at[idx_v], buf_v, sem).wait()
                pltpu.sync_copy(buf_v, out_hbm.at[pl.ds(off, RCH), :])
            return 0

        lax.fori_loop(0, RPW // RCH, body, 0)

    return k(rows, rank)


CB = 512


def _cumsum_body(x_ref, o_ref, carry_ref):
    i = pl.program_id(0)

    @pl.when(i == 0)
    def _():
        carry_ref[...] = jnp.zeros_like(carry_ref)

    blk = x_ref[...]
    ri = lax.broadcasted_iota(jnp.int32, (CB, CB), 0)
    ci = lax.broadcasted_iota(jnp.int32, (CB, CB), 1)
    L = (ri >= ci).astype(jnp.float32)
    c = carry_ref[0:1, :]
    o_ref[...] = lax.dot_general(L, blk, (((1,), (0,)), ((), ())),
                                 preferred_element_type=jnp.float32) + c
    carry_ref[0:1, :] = c + jnp.sum(blk, axis=0, keepdims=True)


def _cumsum(xs):
    return pl.pallas_call(
        _cumsum_body,
        grid=(NP // CB,),
        in_specs=[pl.BlockSpec((CB, D), lambda i: (i, 0))],
        out_specs=pl.BlockSpec((CB, D), lambda i: (i, 0)),
        out_shape=jax.ShapeDtypeStruct((NP, D), jnp.float32),
        scratch_shapes=[pltpu.VMEM((8, D), jnp.float32)],
        compiler_params=pltpu.CompilerParams(
            dimension_semantics=("arbitrary",)),
    )(xs)


def _final_body(x_ref, d_ref, p_ref, r_ref, ucs_ref, a_ref, bp_ref, dp_ref,
                o_ref):
    a = a_ref[...]
    bp = bp_ref[...]
    c1 = bp * ucs_ref[0:1, :]
    c2 = bp * ucs_ref[1:2, :]
    dp = dp_ref[0, 0]
    y = jnp.exp(d_ref[...] * a) * c1 + p_ref[...] * c2
    o = y + r_ref[...] * dp
    mu = jnp.mean(o, axis=1, keepdims=True)
    dev = o - mu
    var = jnp.mean(dev * dev, axis=1, keepdims=True)
    o_ref[...] = x_ref[...] + dev * lax.rsqrt(var + 1e-5)


def _final(x, delta, P, res, ucs, a, bp, dp):
    row = lambda i: (i, 0)
    full = lambda i: (0, 0)
    rspec = pl.BlockSpec((BN, D), row)
    return pl.pallas_call(
        _final_body,
        grid=(N // BN,),
        in_specs=[rspec, rspec, rspec, rspec,
                  pl.BlockSpec((8, D), full), pl.BlockSpec((1, D), full),
                  pl.BlockSpec((1, D), full), pl.BlockSpec((1, 1), full)],
        out_specs=rspec,
        out_shape=jax.ShapeDtypeStruct((N, D), jnp.float32),
        compiler_params=pltpu.CompilerParams(
            dimension_semantics=("arbitrary",)),
    )(x, delta, P, res, ucs, a, bp, dp)


def kernel(x, edge_index, Wl, bl, Wr, Wr1, br1, Wr2, br2, Wproj, A, Bp, Dp,
           Wd, bd):
    aggp, hist = _edge_agg(x, edge_index)

    sinit = jnp.full((NP, 1), -jnp.inf, jnp.float32)
    scores, delta, res, ucs = _dense(
        x, aggp, hist, sinit,
        Wl, bl.reshape(1, D), Wr, Wr1, br1.reshape(1, 32),
        Wr2, br2.reshape(1, 1),
        Wproj[:D], Wproj[2 * D:3 * D], Wproj[3 * D:], Wd, bd.reshape(1, D))

    rank = _rank(scores, scores.reshape(1, NP))
    rank_flat = rank.reshape(NP)

    ds = _permute_rows(delta, rank_flat, scatter=True)
    Ps = _cumsum(ds)
    Ppad = _permute_rows(Ps, rank_flat, scatter=False)

    return _final(x, delta, Ppad, res, ucs,
                  A.reshape(1, D), Bp.reshape(1, D), Dp.reshape(1, 1))
